# Initial kernel scaffold; baseline (speedup 1.0000x reference)
import jax
import jax.numpy as jnp
from jax.experimental import pallas as pl


def kernel(X, E, attr, W_l, b_l, W_r, b_r, att, bias_gat, W_fin, b_fin, ln_gamma, ln_beta):
    raise NotImplementedError("write your pallas kernel here")


# speedup vs baseline: 3.5800x; 3.5800x over previous
import functools

import jax
import jax.numpy as jnp
from jax import lax
from jax.experimental import pallas as pl
from jax.experimental.pallas import tpu as pltpu
from jax.experimental.pallas import tpu_sc as plsc

N = 10000
E = 160000
F_IN = 256
F_OUT = 256
H = 4
HF = H * F_OUT
XW = HF + 128
NEG = 0.2

NC = 2
NS = 16
NW = NC * NS

EPW = E // NW
GA = 8
T_ROWS = 64
ACC_ROWS = 72
TRASH = 66
SB = 2000
N_SLICES = 5
N_PAD = N_SLICES * NW * T_ROWS


def _tc1_body(x_ref, wl_ref, bl_ref, wr_ref, br_ref, att_ref,
              xl_ref, xr_ref, sl_ref):
    x = x_ref[...]
    xl = jnp.dot(x, wl_ref[...], preferred_element_type=jnp.float32) + bl_ref[...]
    xr = jnp.dot(x, wr_ref[...], preferred_element_type=jnp.float32) + br_ref[...]
    bn = x.shape[0]
    ci = lax.broadcasted_iota(jnp.int32, (bn, XW - HF), 1)
    aug = jnp.where((ci % 16 == 0) & (ci < 16 * H), 1.0, 0.0).astype(jnp.float32)
    xl_ref[...] = jnp.concatenate([xl, aug], axis=1)
    xr_ref[...] = xr
    z = jnp.maximum(xl + xr, NEG * (xl + xr))
    cols = []
    for h in range(H):
        zh = z[:, h * F_OUT:(h + 1) * F_OUT] * att_ref[:, h * F_OUT:(h + 1) * F_OUT]
        cols.append(jnp.sum(zh, axis=1, keepdims=True))
    cols.append(jnp.zeros((bn, 16 - H), jnp.float32))
    sl_ref[...] = jnp.concatenate(cols, axis=1)


def _tc1(X, W_l, b_l, W_r, b_r, att_flat):
    BN = 400
    return pl.pallas_call(
        _tc1_body,
        grid=(N // BN,),
        in_specs=[
            pl.BlockSpec((BN, F_IN), lambda i: (i, 0)),
            pl.BlockSpec((F_IN, HF), lambda i: (0, 0)),
            pl.BlockSpec((1, HF), lambda i: (0, 0)),
            pl.BlockSpec((F_IN, HF), lambda i: (0, 0)),
            pl.BlockSpec((1, HF), lambda i: (0, 0)),
            pl.BlockSpec((1, HF), lambda i: (0, 0)),
        ],
        out_specs=[
            pl.BlockSpec((BN, XW), lambda i: (i, 0)),
            pl.BlockSpec((BN, HF), lambda i: (i, 0)),
            pl.BlockSpec((BN, 16), lambda i: (i, 0)),
        ],
        out_shape=[
            jax.ShapeDtypeStruct((N, XW), jnp.float32),
            jax.ShapeDtypeStruct((N, HF), jnp.float32),
            jax.ShapeDtypeStruct((N, 16), jnp.float32),
        ],
    )(X, W_l, b_l.reshape(1, HF), W_r, b_r.reshape(1, HF), att_flat.reshape(1, HF))


def _sca_body(xl_hbm, xr_hbm, src_hbm, dst_hbm, att_hbm,
              part_hbm,
              src_v, dst_v, att_v, xlr_v, xrr_v, out_v):
    wid = lax.axis_index("s") * NC + lax.axis_index("c")
    base = wid * EPW
    pltpu.sync_copy(src_hbm.at[pl.ds(base, EPW)], src_v)
    pltpu.sync_copy(dst_hbm.at[pl.ds(base, EPW)], dst_v)
    pltpu.sync_copy(att_hbm, att_v)

    def chunk(g, _):
        eb = g * GA
        pltpu.sync_copy(xl_hbm.at[src_v.at[pl.ds(eb, GA)]], xlr_v)
        pltpu.sync_copy(xr_hbm.at[dst_v.at[pl.ds(eb, GA)]], xrr_v)

        def edge(e, _):
            for h in range(H):
                acc = jnp.zeros((16,), jnp.float32)
                for j in range(16):
                    off = h * F_OUT + j * 16
                    z = xlr_v[e, pl.ds(off, 16)] + xrr_v[e, pl.ds(off, 16)]
                    acc = acc + jnp.maximum(z, NEG * z) * att_v[pl.ds(off, 16)]
                out_v[e, pl.ds(h * 16, 16)] = acc
            return 0

        lax.fori_loop(0, GA, edge, 0)
        pltpu.sync_copy(out_v, part_hbm.at[pl.ds(base + eb, GA)])
        return 0

    lax.fori_loop(0, EPW // GA, chunk, 0)


def _sca(xl, xr, src, dst, att_flat):
    mesh = plsc.VectorSubcoreMesh(core_axis_name="c", subcore_axis_name="s")
    f = functools.partial(
        pl.kernel,
        mesh=mesh,
        out_type=jax.ShapeDtypeStruct((E, 64), jnp.float32),
        scratch_types=[
            pltpu.VMEM((EPW,), jnp.int32),
            pltpu.VMEM((EPW,), jnp.int32),
            pltpu.VMEM((HF,), jnp.float32),
            pltpu.VMEM((GA, XW), jnp.float32),
            pltpu.VMEM((GA, HF), jnp.float32),
            pltpu.VMEM((GA, 64), jnp.float32),
        ],
    )(_sca_body)
    return f(xl, xr, src, dst, att_flat)


def _tc2_body(part_ref, expv_ref):
    p = part_ref[...]
    cols = []
    for h in range(H):
        l_h = jnp.sum(p[:, h * 16:(h + 1) * 16], axis=1, keepdims=True)
        cols.append(jnp.exp(jnp.minimum(l_h, 60.0)))
    expv_ref[...] = jnp.concatenate(cols, axis=1)


def _tc2(part):
    BE = 4000
    return pl.pallas_call(
        _tc2_body,
        grid=(E // BE,),
        in_specs=[pl.BlockSpec((BE, 64), lambda i: (i, 0))],
        out_specs=pl.BlockSpec((BE, H), lambda i: (i, 0)),
        out_shape=jax.ShapeDtypeStruct((E, H), jnp.float32),
    )(part)


def _scb_body(xl_hbm, expv_hbm, src_hbm, dst_hbm,
              num_hbm,
              dstblk_v, gid1_v, gbuf_v, sidx_v, ldst_v, evidx_v, evgot_v,
              rows_v, acc_v):
    c = lax.axis_index("c")
    s = lax.axis_index("s")
    wid = s * NC + c
    zeros16 = jnp.zeros((16,), jnp.float32)
    iota = lax.iota(jnp.int32, 16)
    ldst_v[pl.ds(16, 16)] = jnp.full((16,), TRASH, jnp.int32)

    def do_pass(p, _):
        lo = p * (NW * T_ROWS) + wid * T_ROWS

        def zacc(r, _):
            for j in range(XW // 16):
                acc_v[r, pl.ds(j * 16, 16)] = zeros16
            return 0
        lax.fori_loop(0, ACC_ROWS, zacc, 0)

        def blk(b, _):
            pltpu.sync_copy(dst_hbm.at[pl.ds(b * SB, SB)], dstblk_v)

            def scan(i, cnt):
                d16 = dstblk_v[pl.ds(i * 16, 16)]
                m = (d16 >= lo) & (d16 < lo + T_ROWS)
                gids = b * SB + i * 16 + iota
                plsc.store_compressed(gid1_v.at[pl.ds(cnt, 16)], gids, mask=m)
                pc = plsc.all_reduce_population_count(m)
                return cnt + pc[0]
            cnt = lax.fori_loop(0, SB // 16, scan, 0)
            nch = lax.div(cnt + 15, 16)

            def chunk(k, _):
                valid = (k * 16 + iota) < cnt
                g16 = jnp.where(valid, gid1_v[pl.ds(k * 16, 16)], b * SB)
                gbuf_v[pl.ds(0, 16)] = g16
                pltpu.sync_copy(src_hbm.at[gbuf_v], sidx_v)
                for h in range(H):
                    plsc.store_scatter(evidx_v, [iota * 4 + h], g16 * 4 + h)
                pltpu.sync_copy(xl_hbm.at[sidx_v], rows_v)
                pltpu.sync_copy(expv_hbm.at[evidx_v], evgot_v.at[pl.ds(0, 64)])
                d16 = plsc.load_gather(dstblk_v, [g16 - b * SB])
                ldst_v[pl.ds(0, 16)] = jnp.where(valid, d16 - lo, TRASH)

                def edge(e, _):
                    evrow = evgot_v[pl.ds(e * 4, 16)]
                    ld = ldst_v[pl.ds(e, 16)][0]
                    for h in range(H):
                        evs = evrow[h]
                        for j in range(16):
                            off = h * F_OUT + j * 16
                            plsc.addupdate(acc_v.at[ld, pl.ds(off, 16)],
                                           rows_v[e, pl.ds(off, 16)] * evs)
                        offz = HF + h * 16
                        plsc.addupdate(acc_v.at[ld, pl.ds(offz, 16)],
                                       rows_v[e, pl.ds(offz, 16)] * evs)
                    return 0
                lax.fori_loop(0, 16, edge, 0)
                return 0
            lax.fori_loop(0, nch, chunk, 0)
            return 0
        lax.fori_loop(0, E // SB, blk, 0)

        pltpu.sync_copy(acc_v.at[pl.ds(0, T_ROWS)],
                        num_hbm.at[pl.ds(lo, T_ROWS)])
        return 0

    lax.fori_loop(0, N_SLICES, do_pass, 0)


def _scb(xl, expv_flat, src, dst):
    mesh = plsc.VectorSubcoreMesh(core_axis_name="c", subcore_axis_name="s")
    f = functools.partial(
        pl.kernel,
        mesh=mesh,
        compiler_params=pltpu.CompilerParams(needs_layout_passes=False),
        out_type=jax.ShapeDtypeStruct((N_PAD, XW), jnp.float32),
        scratch_types=[
            pltpu.VMEM((SB,), jnp.int32),
            pltpu.VMEM((SB + 16,), jnp.int32),
            pltpu.VMEM((16,), jnp.int32),
            pltpu.VMEM((16,), jnp.int32),
            pltpu.VMEM((32,), jnp.int32),
            pltpu.VMEM((64,), jnp.int32),
            pltpu.VMEM((80,), jnp.float32),
            pltpu.VMEM((16, XW), jnp.float32),
            pltpu.VMEM((ACC_ROWS, XW), jnp.float32),
        ],
    )(_scb_body)
    return f(xl, expv_flat, src, dst)


def _tc3_body(num_ref, sl_ref, xl_ref, bg_ref, wf_ref, bf_ref, g_ref,
              b_ref, out_ref):
    cols = []
    for h in range(H):
        es = jnp.exp(jnp.minimum(sl_ref[:, h:h + 1], 60.0))
        znorm = num_ref[:, HF + 16 * h:HF + 16 * h + 1] + es + 1e-16
        seg = (num_ref[:, h * F_OUT:(h + 1) * F_OUT]
               + es * xl_ref[:, h * F_OUT:(h + 1) * F_OUT]) / znorm
        cols.append(seg)
    hg = jnp.concatenate(cols, axis=1) + bg_ref[...]
    hlin = jnp.dot(hg, wf_ref[...], preferred_element_type=jnp.float32) + bf_ref[...]
    mu = jnp.mean(hlin, axis=1, keepdims=True)
    var = jnp.mean((hlin - mu) ** 2, axis=1, keepdims=True)
    out_ref[...] = (hlin - mu) * lax.rsqrt(var + 1e-5) * g_ref[...] + b_ref[...]


def _tc3(num, sl, xl, bias_gat, W_fin, b_fin, ln_gamma, ln_beta):
    BN = 400
    return pl.pallas_call(
        _tc3_body,
        grid=(N // BN,),
        in_specs=[
            pl.BlockSpec((BN, XW), lambda i: (i, 0)),
            pl.BlockSpec((BN, 16), lambda i: (i, 0)),
            pl.BlockSpec((BN, XW), lambda i: (i, 0)),
            pl.BlockSpec((1, HF), lambda i: (0, 0)),
            pl.BlockSpec((HF, F_OUT), lambda i: (0, 0)),
            pl.BlockSpec((1, F_OUT), lambda i: (0, 0)),
            pl.BlockSpec((1, F_OUT), lambda i: (0, 0)),
            pl.BlockSpec((1, F_OUT), lambda i: (0, 0)),
        ],
        out_specs=pl.BlockSpec((BN, F_OUT), lambda i: (i, 0)),
        out_shape=jax.ShapeDtypeStruct((N, F_OUT), jnp.float32),
    )(num, sl, xl, bias_gat.reshape(1, HF), W_fin, b_fin.reshape(1, F_OUT),
      ln_gamma.reshape(1, F_OUT), ln_beta.reshape(1, F_OUT))


def kernel(X, E_idx, attr, W_l, b_l, W_r, b_r, att, bias_gat, W_fin, b_fin,
           ln_gamma, ln_beta):
    att_flat = att.reshape(HF)
    src = E_idx[0]
    dst = E_idx[1]
    xl, xr, sl = _tc1(X, W_l, b_l, W_r, b_r, att_flat)
    part = _sca(xl, xr, src, dst, att_flat)
    expv = _tc2(part)
    num = _scb(xl, expv.reshape(E * H), src, dst)
    h = _tc3(num[:N], sl, xl, bias_gat, W_fin, b_fin, ln_gamma, ln_beta)
    return (h, E_idx, attr)


# speedup vs baseline: 4.8407x; 1.3521x over previous
import functools

import jax
import jax.numpy as jnp
from jax import lax
from jax.experimental import pallas as pl
from jax.experimental.pallas import tpu as pltpu
from jax.experimental.pallas import tpu_sc as plsc

N = 10000
E = 160000
F_IN = 256
F_OUT = 256
H = 4
HF = H * F_OUT
XW = HF + 128
NEG = 0.2

NC = 2
NS = 16
NW = NC * NS

EPW = E // NW
GA = 8
T_ROWS = 64
ACC_ROWS = 72
TRASH = 66
SB = 1600
GB = 32
N_SLICES = 5
N_PAD = N_SLICES * NW * T_ROWS


def _tc1_body(x_ref, wl_ref, bl_ref, wr_ref, br_ref, att_ref,
              xl_ref, xr_ref, sl_ref):
    x = x_ref[...]
    xl = jnp.dot(x, wl_ref[...], preferred_element_type=jnp.float32) + bl_ref[...]
    xr = jnp.dot(x, wr_ref[...], preferred_element_type=jnp.float32) + br_ref[...]
    bn = x.shape[0]
    ci = lax.broadcasted_iota(jnp.int32, (bn, XW - HF), 1)
    aug = jnp.where((ci % 16 == 0) & (ci < 16 * H), 1.0, 0.0).astype(jnp.float32)
    xl_ref[...] = jnp.concatenate([xl, aug], axis=1)
    xr_ref[...] = xr
    z = jnp.maximum(xl + xr, NEG * (xl + xr))
    cols = []
    for h in range(H):
        zh = z[:, h * F_OUT:(h + 1) * F_OUT] * att_ref[:, h * F_OUT:(h + 1) * F_OUT]
        cols.append(jnp.sum(zh, axis=1, keepdims=True))
    cols.append(jnp.zeros((bn, 16 - H), jnp.float32))
    sl_ref[...] = jnp.concatenate(cols, axis=1)


def _tc1(X, W_l, b_l, W_r, b_r, att_flat):
    BN = 400
    return pl.pallas_call(
        _tc1_body,
        grid=(N // BN,),
        in_specs=[
            pl.BlockSpec((BN, F_IN), lambda i: (i, 0)),
            pl.BlockSpec((F_IN, HF), lambda i: (0, 0)),
            pl.BlockSpec((1, HF), lambda i: (0, 0)),
            pl.BlockSpec((F_IN, HF), lambda i: (0, 0)),
            pl.BlockSpec((1, HF), lambda i: (0, 0)),
            pl.BlockSpec((1, HF), lambda i: (0, 0)),
        ],
        out_specs=[
            pl.BlockSpec((BN, XW), lambda i: (i, 0)),
            pl.BlockSpec((BN, HF), lambda i: (i, 0)),
            pl.BlockSpec((BN, 16), lambda i: (i, 0)),
        ],
        out_shape=[
            jax.ShapeDtypeStruct((N, XW), jnp.float32),
            jax.ShapeDtypeStruct((N, HF), jnp.float32),
            jax.ShapeDtypeStruct((N, 16), jnp.float32),
        ],
    )(X, W_l, b_l.reshape(1, HF), W_r, b_r.reshape(1, HF), att_flat.reshape(1, HF))


def _sca_body(xl_hbm, xr_hbm, src_hbm, dst_hbm, att_hbm,
              part_hbm,
              src_v, dst_v, att_v, xlr_v, xrr_v, out_v):
    wid = lax.axis_index("s") * NC + lax.axis_index("c")
    base = wid * EPW
    pltpu.sync_copy(src_hbm.at[pl.ds(base, EPW)], src_v)
    pltpu.sync_copy(dst_hbm.at[pl.ds(base, EPW)], dst_v)
    pltpu.sync_copy(att_hbm, att_v)

    def chunk(g, _):
        eb = g * GA
        pltpu.sync_copy(xl_hbm.at[src_v.at[pl.ds(eb, GA)]], xlr_v)
        pltpu.sync_copy(xr_hbm.at[dst_v.at[pl.ds(eb, GA)]], xrr_v)

        def edge(e, _):
            for h in range(H):
                acc = jnp.zeros((16,), jnp.float32)
                for j in range(16):
                    off = h * F_OUT + j * 16
                    z = xlr_v[e, pl.ds(off, 16)] + xrr_v[e, pl.ds(off, 16)]
                    acc = acc + jnp.maximum(z, NEG * z) * att_v[pl.ds(off, 16)]
                out_v[e, pl.ds(h * 16, 16)] = acc
            return 0

        lax.fori_loop(0, GA, edge, 0)
        pltpu.sync_copy(out_v, part_hbm.at[pl.ds(base + eb, GA)])
        return 0

    lax.fori_loop(0, EPW // GA, chunk, 0)


def _sca(xl, xr, src, dst, att_flat):
    mesh = plsc.VectorSubcoreMesh(core_axis_name="c", subcore_axis_name="s")
    f = functools.partial(
        pl.kernel,
        mesh=mesh,
        out_type=jax.ShapeDtypeStruct((E, 64), jnp.float32),
        scratch_types=[
            pltpu.VMEM((EPW,), jnp.int32),
            pltpu.VMEM((EPW,), jnp.int32),
            pltpu.VMEM((HF,), jnp.float32),
            pltpu.VMEM((GA, XW), jnp.float32),
            pltpu.VMEM((GA, HF), jnp.float32),
            pltpu.VMEM((GA, 64), jnp.float32),
        ],
    )(_sca_body)
    return f(xl, xr, src, dst, att_flat)


def _tc2_body(part_ref, expv_ref):
    p = part_ref[...]
    cols = []
    for h in range(H):
        l_h = jnp.sum(p[:, h * 16:(h + 1) * 16], axis=1, keepdims=True)
        cols.append(jnp.exp(jnp.minimum(l_h, 60.0)))
    expv_ref[...] = jnp.concatenate(cols, axis=1)


def _tc2(part):
    BE = 4000
    return pl.pallas_call(
        _tc2_body,
        grid=(E // BE,),
        in_specs=[pl.BlockSpec((BE, 64), lambda i: (i, 0))],
        out_specs=pl.BlockSpec((BE, H), lambda i: (i, 0)),
        out_shape=jax.ShapeDtypeStruct((E, H), jnp.float32),
    )(part)


def _scb_body(xl_hbm, expv_hbm, src_hbm, dst_hbm,
              num_hbm,
              blk0_v, blk1_v, gidl_v, ldl_v, gbuf_v, sidx_v, ldst_v,
              evidx_v, evgot_v, rows_v, acc_v, semd, seme):
    c = lax.axis_index("c")
    s = lax.axis_index("s")
    wid = s * NC + c
    zeros16 = jnp.zeros((16,), jnp.float32)
    iota = lax.iota(jnp.int32, 16)
    ldst_v[pl.ds(GB, 16)] = jnp.full((16,), TRASH, jnp.int32)
    NB = E // SB

    def do_pass(p, _):
        lo = p * (NW * T_ROWS) + wid * T_ROWS

        def zacc(r, _):
            for j in range(XW // 16):
                acc_v[r, pl.ds(j * 16, 16)] = zeros16
            return 0
        lax.fori_loop(0, ACC_ROWS, zacc, 0)

        def do_chunk(base, avail):
            gs = []
            for q in range(GB // 16):
                gq = gidl_v[pl.ds(base + q * 16, 16)]
                gq = jnp.where((q * 16 + iota) < avail, gq, 0)
                gs.append(gq)
                gbuf_v[pl.ds(q * 16, 16)] = gq
                ldq = ldl_v[pl.ds(base + q * 16, 16)]
                ldst_v[pl.ds(q * 16, 16)] = jnp.where(
                    (q * 16 + iota) < avail, ldq, TRASH)
            for q in range(GB // 16):
                for h in range(H):
                    plsc.store_scatter(
                        evidx_v.at[pl.ds(q * 64, 64)],
                        [iota * 4 + h], gs[q] * 4 + h)
            pltpu.async_copy(expv_hbm.at[evidx_v], evgot_v.at[pl.ds(0, 4 * GB)],
                             seme)
            pltpu.sync_copy(src_hbm.at[gbuf_v], sidx_v)
            pltpu.sync_copy(xl_hbm.at[sidx_v], rows_v)
            pltpu.make_async_copy(expv_hbm.at[evidx_v],
                                  evgot_v.at[pl.ds(0, 4 * GB)], seme).wait()

            def edge(e, _):
                evrow = evgot_v[pl.ds(e * 4, 16)]
                ld = ldst_v[pl.ds(e, 16)][0]
                for h in range(H):
                    evs = evrow[h]
                    for j in range(16):
                        off = h * F_OUT + j * 16
                        plsc.addupdate(acc_v.at[ld, pl.ds(off, 16)],
                                       rows_v[e, pl.ds(off, 16)] * evs)
                    offz = HF + h * 16
                    plsc.addupdate(acc_v.at[ld, pl.ds(offz, 16)],
                                   rows_v[e, pl.ds(offz, 16)] * evs)
                return 0
            lax.fori_loop(0, GB, edge, 0)

        pltpu.async_copy(dst_hbm.at[pl.ds(0, SB)], blk0_v, semd)

        def blk(b, rem):
            def scan_into(blk_v):
                pltpu.make_async_copy(dst_hbm.at[pl.ds(b * SB, SB)], blk_v,
                                      semd).wait()

                def scan(i, cnt):
                    d16 = blk_v[pl.ds(i * 16, 16)]
                    m = (d16 >= lo) & (d16 < lo + T_ROWS)
                    gids = b * SB + i * 16 + iota
                    plsc.store_compressed(gidl_v.at[pl.ds(cnt, 16)], gids,
                                          mask=m)
                    plsc.store_compressed(ldl_v.at[pl.ds(cnt, 16)], d16 - lo,
                                          mask=m)
                    pc = plsc.all_reduce_population_count(m)
                    return cnt + pc[0]
                return lax.fori_loop(0, SB // 16, scan, rem)

            @pl.when(b % 2 == 0)
            def _():
                @pl.when(b + 1 < NB)
                def _():
                    pltpu.async_copy(dst_hbm.at[pl.ds((b + 1) * SB, SB)],
                                     blk1_v, semd)
            @pl.when(b % 2 == 1)
            def _():
                @pl.when(b + 1 < NB)
                def _():
                    pltpu.async_copy(dst_hbm.at[pl.ds((b + 1) * SB, SB)],
                                     blk0_v, semd)
            avail0 = lax.cond(b % 2 == 0,
                              lambda: scan_into(blk0_v),
                              lambda: scan_into(blk1_v))

            nfull = lax.div(avail0, GB)

            def chunk(k, _):
                do_chunk(k * GB, GB)
                return 0
            lax.fori_loop(0, nfull, chunk, 0)

            rem2 = avail0 - nfull * GB
            tb = nfull * GB
            for q in range(GB // 16):
                gq = gidl_v[pl.ds(tb + q * 16, 16)]
                lq = ldl_v[pl.ds(tb + q * 16, 16)]
                gidl_v[pl.ds(q * 16, 16)] = gq
                ldl_v[pl.ds(q * 16, 16)] = lq
            return rem2

        rem = lax.fori_loop(0, NB, blk, 0)

        @pl.when(rem > 0)
        def _():
            do_chunk(0, rem)

        pltpu.sync_copy(acc_v.at[pl.ds(0, T_ROWS)],
                        num_hbm.at[pl.ds(lo, T_ROWS)])
        return 0

    lax.fori_loop(0, N_SLICES, do_pass, 0)


def _scb(xl, expv_flat, src, dst):
    mesh = plsc.VectorSubcoreMesh(core_axis_name="c", subcore_axis_name="s")
    f = functools.partial(
        pl.kernel,
        mesh=mesh,
        compiler_params=pltpu.CompilerParams(needs_layout_passes=False),
        out_type=jax.ShapeDtypeStruct((N_PAD, XW), jnp.float32),
        scratch_types=[
            pltpu.VMEM((SB,), jnp.int32),
            pltpu.VMEM((SB,), jnp.int32),
            pltpu.VMEM((SB + GB + 32,), jnp.int32),
            pltpu.VMEM((SB + GB + 32,), jnp.int32),
            pltpu.VMEM((GB,), jnp.int32),
            pltpu.VMEM((GB,), jnp.int32),
            pltpu.VMEM((GB + 16,), jnp.int32),
            pltpu.VMEM((4 * GB,), jnp.int32),
            pltpu.VMEM((4 * GB + 16,), jnp.float32),
            pltpu.VMEM((GB, XW), jnp.float32),
            pltpu.VMEM((ACC_ROWS, XW), jnp.float32),
            pltpu.SemaphoreType.DMA,
            pltpu.SemaphoreType.DMA,
        ],
    )(_scb_body)
    return f(xl, expv_flat, src, dst)


def _tc3_body(num_ref, sl_ref, xl_ref, bg_ref, wf_ref, bf_ref, g_ref,
              b_ref, out_ref):
    cols = []
    for h in range(H):
        es = jnp.exp(jnp.minimum(sl_ref[:, h:h + 1], 60.0))
        znorm = num_ref[:, HF + 16 * h:HF + 16 * h + 1] + es + 1e-16
        seg = (num_ref[:, h * F_OUT:(h + 1) * F_OUT]
               + es * xl_ref[:, h * F_OUT:(h + 1) * F_OUT]) / znorm
        cols.append(seg)
    hg = jnp.concatenate(cols, axis=1) + bg_ref[...]
    hlin = jnp.dot(hg, wf_ref[...], preferred_element_type=jnp.float32) + bf_ref[...]
    mu = jnp.mean(hlin, axis=1, keepdims=True)
    var = jnp.mean((hlin - mu) ** 2, axis=1, keepdims=True)
    out_ref[...] = (hlin - mu) * lax.rsqrt(var + 1e-5) * g_ref[...] + b_ref[...]


def _tc3(num, sl, xl, bias_gat, W_fin, b_fin, ln_gamma, ln_beta):
    BN = 400
    return pl.pallas_call(
        _tc3_body,
        grid=(N // BN,),
        in_specs=[
            pl.BlockSpec((BN, XW), lambda i: (i, 0)),
            pl.BlockSpec((BN, 16), lambda i: (i, 0)),
            pl.BlockSpec((BN, XW), lambda i: (i, 0)),
            pl.BlockSpec((1, HF), lambda i: (0, 0)),
            pl.BlockSpec((HF, F_OUT), lambda i: (0, 0)),
            pl.BlockSpec((1, F_OUT), lambda i: (0, 0)),
            pl.BlockSpec((1, F_OUT), lambda i: (0, 0)),
            pl.BlockSpec((1, F_OUT), lambda i: (0, 0)),
        ],
        out_specs=pl.BlockSpec((BN, F_OUT), lambda i: (i, 0)),
        out_shape=jax.ShapeDtypeStruct((N, F_OUT), jnp.float32),
    )(num, sl, xl, bias_gat.reshape(1, HF), W_fin, b_fin.reshape(1, F_OUT),
      ln_gamma.reshape(1, F_OUT), ln_beta.reshape(1, F_OUT))


def kernel(X, E_idx, attr, W_l, b_l, W_r, b_r, att, bias_gat, W_fin, b_fin,
           ln_gamma, ln_beta):
    att_flat = att.reshape(HF)
    src = E_idx[0]
    dst = E_idx[1]
    xl, xr, sl = _tc1(X, W_l, b_l, W_r, b_r, att_flat)
    part = _sca(xl, xr, src, dst, att_flat)
    expv = _tc2(part)
    num = _scb(xl, expv.reshape(E * H), src, dst)
    h = _tc3(num[:N], sl, xl, bias_gat, W_fin, b_fin, ln_gamma, ln_beta)
    return (h, E_idx, attr)


# speedup vs baseline: 6.2651x; 1.2943x over previous
import functools

import jax
import jax.numpy as jnp
from jax import lax
from jax.experimental import pallas as pl
from jax.experimental.pallas import tpu as pltpu
from jax.experimental.pallas import tpu_sc as plsc

N = 10000
E = 160000
F_IN = 256
F_OUT = 256
H = 4
HF = H * F_OUT
XW = HF + 128
NEG = 0.2

NC = 2
NS = 16
NW = NC * NS

EPW = E // NW
GA = 8
T_ROWS = 64
ACC_ROWS = 72
TRASH = 66
SB = 1600
GB = 32
N_SLICES = 5
N_PAD = N_SLICES * NW * T_ROWS


def _tc1_body(x_ref, wl_ref, bl_ref, wr_ref, br_ref, att_ref,
              xl_ref, xr_ref, sl_ref):
    x = x_ref[...]
    xl = jnp.dot(x, wl_ref[...], preferred_element_type=jnp.float32) + bl_ref[...]
    xr = jnp.dot(x, wr_ref[...], preferred_element_type=jnp.float32) + br_ref[...]
    bn = x.shape[0]
    ci = lax.broadcasted_iota(jnp.int32, (bn, XW - HF), 1)
    aug = jnp.where((ci % 16 == 0) & (ci < 16 * H), 1.0, 0.0).astype(jnp.float32)
    xl_ref[...] = jnp.concatenate([xl, aug], axis=1)
    xr_ref[...] = xr
    z = jnp.maximum(xl + xr, NEG * (xl + xr))
    cols = []
    for h in range(H):
        zh = z[:, h * F_OUT:(h + 1) * F_OUT] * att_ref[:, h * F_OUT:(h + 1) * F_OUT]
        cols.append(jnp.sum(zh, axis=1, keepdims=True))
    cols.append(jnp.zeros((bn, 16 - H), jnp.float32))
    sl_ref[...] = jnp.concatenate(cols, axis=1)


def _tc1(X, W_l, b_l, W_r, b_r, att_flat):
    BN = 400
    return pl.pallas_call(
        _tc1_body,
        grid=(N // BN,),
        in_specs=[
            pl.BlockSpec((BN, F_IN), lambda i: (i, 0)),
            pl.BlockSpec((F_IN, HF), lambda i: (0, 0)),
            pl.BlockSpec((1, HF), lambda i: (0, 0)),
            pl.BlockSpec((F_IN, HF), lambda i: (0, 0)),
            pl.BlockSpec((1, HF), lambda i: (0, 0)),
            pl.BlockSpec((1, HF), lambda i: (0, 0)),
        ],
        out_specs=[
            pl.BlockSpec((BN, XW), lambda i: (i, 0)),
            pl.BlockSpec((BN, HF), lambda i: (i, 0)),
            pl.BlockSpec((BN, 16), lambda i: (i, 0)),
        ],
        out_shape=[
            jax.ShapeDtypeStruct((N, XW), jnp.float32),
            jax.ShapeDtypeStruct((N, HF), jnp.float32),
            jax.ShapeDtypeStruct((N, 16), jnp.float32),
        ],
    )(X, W_l, b_l.reshape(1, HF), W_r, b_r.reshape(1, HF), att_flat.reshape(1, HF))


def _sca_body(xl_hbm, xr_hbm, src_hbm, dst_hbm, att_hbm,
              part_hbm,
              src_v, dst_v, att_v, xlr0_v, xlr1_v, xrr0_v, xrr1_v,
              out0_v, out1_v, semg0, semg1, semo0, semo1):
    wid = lax.axis_index("s") * NC + lax.axis_index("c")
    base = wid * EPW
    NCH = EPW // GA
    pltpu.sync_copy(src_hbm.at[pl.ds(base, EPW)], src_v)
    pltpu.sync_copy(dst_hbm.at[pl.ds(base, EPW)], dst_v)
    pltpu.sync_copy(att_hbm, att_v)

    def fire(g, xlr, xrr, semg):
        eb = g * GA
        pltpu.async_copy(xl_hbm.at[src_v.at[pl.ds(eb, GA)]], xlr, semg)
        pltpu.async_copy(xr_hbm.at[dst_v.at[pl.ds(eb, GA)]], xrr, semg)

    def half(g, xlr, xrr, out_v, semg, semo):
        eb = g * GA
        pltpu.make_async_copy(xl_hbm.at[src_v.at[pl.ds(eb, GA)]], xlr,
                              semg).wait()
        pltpu.make_async_copy(xr_hbm.at[dst_v.at[pl.ds(eb, GA)]], xrr,
                              semg).wait()

        @pl.when(g >= 2)
        def _():
            pltpu.make_async_copy(out_v, part_hbm.at[pl.ds(base, GA)],
                                  semo).wait()

        def edge(e, _):
            for h in range(H):
                acc = jnp.zeros((16,), jnp.float32)
                for j in range(16):
                    off = h * F_OUT + j * 16
                    z = xlr[e, pl.ds(off, 16)] + xrr[e, pl.ds(off, 16)]
                    acc = acc + jnp.maximum(z, NEG * z) * att_v[pl.ds(off, 16)]
                out_v[e, pl.ds(h * 16, 16)] = acc
            return 0
        lax.fori_loop(0, GA, edge, 0)

        pltpu.async_copy(out_v, part_hbm.at[pl.ds(base + eb, GA)], semo)

        @pl.when(g + 2 < NCH)
        def _():
            fire(g + 2, xlr, xrr, semg)

    fire(0, xlr0_v, xrr0_v, semg0)
    fire(1, xlr1_v, xrr1_v, semg1)

    def chunk(g, _):
        @pl.when(g % 2 == 0)
        def _():
            half(g, xlr0_v, xrr0_v, out0_v, semg0, semo0)
        @pl.when(g % 2 == 1)
        def _():
            half(g, xlr1_v, xrr1_v, out1_v, semg1, semo1)
        return 0
    lax.fori_loop(0, NCH, chunk, 0)

    pltpu.make_async_copy(out0_v, part_hbm.at[pl.ds(base, GA)], semo0).wait()
    pltpu.make_async_copy(out1_v, part_hbm.at[pl.ds(base, GA)], semo1).wait()


def _sca(xl, xr, src, dst, att_flat):
    mesh = plsc.VectorSubcoreMesh(core_axis_name="c", subcore_axis_name="s")
    f = functools.partial(
        pl.kernel,
        mesh=mesh,
        compiler_params=pltpu.CompilerParams(needs_layout_passes=False),
        out_type=jax.ShapeDtypeStruct((E, 64), jnp.float32),
        scratch_types=[
            pltpu.VMEM((EPW,), jnp.int32),
            pltpu.VMEM((EPW,), jnp.int32),
            pltpu.VMEM((HF,), jnp.float32),
            pltpu.VMEM((GA, XW), jnp.float32),
            pltpu.VMEM((GA, XW), jnp.float32),
            pltpu.VMEM((GA, HF), jnp.float32),
            pltpu.VMEM((GA, HF), jnp.float32),
            pltpu.VMEM((GA, 64), jnp.float32),
            pltpu.VMEM((GA, 64), jnp.float32),
            pltpu.SemaphoreType.DMA,
            pltpu.SemaphoreType.DMA,
            pltpu.SemaphoreType.DMA,
            pltpu.SemaphoreType.DMA,
        ],
    )(_sca_body)
    return f(xl, xr, src, dst, att_flat)


def _tc2_body(part_ref, expv_ref):
    p = part_ref[...]
    cols = []
    for h in range(H):
        l_h = jnp.sum(p[:, h * 16:(h + 1) * 16], axis=1, keepdims=True)
        cols.append(jnp.exp(jnp.minimum(l_h, 60.0)))
    expv_ref[...] = jnp.concatenate(cols, axis=1)


def _tc2(part):
    BE = 4000
    return pl.pallas_call(
        _tc2_body,
        grid=(E // BE,),
        in_specs=[pl.BlockSpec((BE, 64), lambda i: (i, 0))],
        out_specs=pl.BlockSpec((BE, H), lambda i: (i, 0)),
        out_shape=jax.ShapeDtypeStruct((E, H), jnp.float32),
    )(part)


def _scb_body(xl_hbm, expv_hbm, src_hbm, dst_hbm,
              num_hbm,
              blk0_v, blk1_v, gidl_v, ldl_v, gbuf_v, sidx_v, ldst_v,
              evidx_v, evgot_v, rows_v, acc_v, semd, seme):
    c = lax.axis_index("c")
    s = lax.axis_index("s")
    wid = s * NC + c
    zeros16 = jnp.zeros((16,), jnp.float32)
    iota = lax.iota(jnp.int32, 16)
    ldst_v[pl.ds(GB, 16)] = jnp.full((16,), TRASH, jnp.int32)
    NB = E // SB

    def do_pass(p, _):
        lo = p * (NW * T_ROWS) + wid * T_ROWS

        def zacc(r, _):
            for j in range(XW // 16):
                acc_v[r, pl.ds(j * 16, 16)] = zeros16
            return 0
        lax.fori_loop(0, ACC_ROWS, zacc, 0)

        def do_chunk(base, avail):
            gs = []
            for q in range(GB // 16):
                gq = gidl_v[pl.ds(base + q * 16, 16)]
                gq = jnp.where((q * 16 + iota) < avail, gq, 0)
                gs.append(gq)
                gbuf_v[pl.ds(q * 16, 16)] = gq
                ldq = ldl_v[pl.ds(base + q * 16, 16)]
                ldst_v[pl.ds(q * 16, 16)] = jnp.where(
                    (q * 16 + iota) < avail, ldq, TRASH)
            for q in range(GB // 16):
                for h in range(H):
                    plsc.store_scatter(
                        evidx_v.at[pl.ds(q * 64, 64)],
                        [iota * 4 + h], gs[q] * 4 + h)
            pltpu.async_copy(expv_hbm.at[evidx_v], evgot_v.at[pl.ds(0, 4 * GB)],
                             seme)
            pltpu.sync_copy(src_hbm.at[gbuf_v], sidx_v)
            pltpu.sync_copy(xl_hbm.at[sidx_v], rows_v)
            pltpu.make_async_copy(expv_hbm.at[evidx_v],
                                  evgot_v.at[pl.ds(0, 4 * GB)], seme).wait()

            def edge(e, _):
                evrow = evgot_v[pl.ds(e * 4, 16)]
                ld = ldst_v[pl.ds(e, 16)][0]
                for h in range(H):
                    evs = evrow[h]
                    for j in range(16):
                        off = h * F_OUT + j * 16
                        plsc.addupdate(acc_v.at[ld, pl.ds(off, 16)],
                                       rows_v[e, pl.ds(off, 16)] * evs)
                    offz = HF + h * 16
                    plsc.addupdate(acc_v.at[ld, pl.ds(offz, 16)],
                                   rows_v[e, pl.ds(offz, 16)] * evs)
                return 0
            lax.fori_loop(0, GB, edge, 0)

        pltpu.async_copy(dst_hbm.at[pl.ds(0, SB)], blk0_v, semd)

        def blk(b, rem):
            def scan_into(blk_v):
                pltpu.make_async_copy(dst_hbm.at[pl.ds(b * SB, SB)], blk_v,
                                      semd).wait()

                def scan(i, cnt):
                    d16 = blk_v[pl.ds(i * 16, 16)]
                    m = (d16 >= lo) & (d16 < lo + T_ROWS)
                    gids = b * SB + i * 16 + iota
                    plsc.store_compressed(gidl_v.at[pl.ds(cnt, 16)], gids,
                                          mask=m)
                    plsc.store_compressed(ldl_v.at[pl.ds(cnt, 16)], d16 - lo,
                                          mask=m)
                    pc = plsc.all_reduce_population_count(m)
                    return cnt + pc[0]
                return lax.fori_loop(0, SB // 16, scan, rem)

            @pl.when(b % 2 == 0)
            def _():
                @pl.when(b + 1 < NB)
                def _():
                    pltpu.async_copy(dst_hbm.at[pl.ds((b + 1) * SB, SB)],
                                     blk1_v, semd)
            @pl.when(b % 2 == 1)
            def _():
                @pl.when(b + 1 < NB)
                def _():
                    pltpu.async_copy(dst_hbm.at[pl.ds((b + 1) * SB, SB)],
                                     blk0_v, semd)
            avail0 = lax.cond(b % 2 == 0,
                              lambda: scan_into(blk0_v),
                              lambda: scan_into(blk1_v))

            nfull = lax.div(avail0, GB)

            def chunk(k, _):
                do_chunk(k * GB, GB)
                return 0
            lax.fori_loop(0, nfull, chunk, 0)

            rem2 = avail0 - nfull * GB
            tb = nfull * GB
            for q in range(GB // 16):
                gq = gidl_v[pl.ds(tb + q * 16, 16)]
                lq = ldl_v[pl.ds(tb + q * 16, 16)]
                gidl_v[pl.ds(q * 16, 16)] = gq
                ldl_v[pl.ds(q * 16, 16)] = lq
            return rem2

        rem = lax.fori_loop(0, NB, blk, 0)

        @pl.when(rem > 0)
        def _():
            do_chunk(0, rem)

        pltpu.sync_copy(acc_v.at[pl.ds(0, T_ROWS)],
                        num_hbm.at[pl.ds(lo, T_ROWS)])
        return 0

    lax.fori_loop(0, N_SLICES, do_pass, 0)


def _scb(xl, expv_flat, src, dst):
    mesh = plsc.VectorSubcoreMesh(core_axis_name="c", subcore_axis_name="s")
    f = functools.partial(
        pl.kernel,
        mesh=mesh,
        compiler_params=pltpu.CompilerParams(needs_layout_passes=False),
        out_type=jax.ShapeDtypeStruct((N_PAD, XW), jnp.float32),
        scratch_types=[
            pltpu.VMEM((SB,), jnp.int32),
            pltpu.VMEM((SB,), jnp.int32),
            pltpu.VMEM((SB + GB + 32,), jnp.int32),
            pltpu.VMEM((SB + GB + 32,), jnp.int32),
            pltpu.VMEM((GB,), jnp.int32),
            pltpu.VMEM((GB,), jnp.int32),
            pltpu.VMEM((GB + 16,), jnp.int32),
            pltpu.VMEM((4 * GB,), jnp.int32),
            pltpu.VMEM((4 * GB + 16,), jnp.float32),
            pltpu.VMEM((GB, XW), jnp.float32),
            pltpu.VMEM((ACC_ROWS, XW), jnp.float32),
            pltpu.SemaphoreType.DMA,
            pltpu.SemaphoreType.DMA,
        ],
    )(_scb_body)
    return f(xl, expv_flat, src, dst)


def _tc3_body(num_ref, sl_ref, xl_ref, bg_ref, wf_ref, bf_ref, g_ref,
              b_ref, out_ref):
    cols = []
    for h in range(H):
        es = jnp.exp(jnp.minimum(sl_ref[:, h:h + 1], 60.0))
        znorm = num_ref[:, HF + 16 * h:HF + 16 * h + 1] + es + 1e-16
        seg = (num_ref[:, h * F_OUT:(h + 1) * F_OUT]
               + es * xl_ref[:, h * F_OUT:(h + 1) * F_OUT]) / znorm
        cols.append(seg)
    hg = jnp.concatenate(cols, axis=1) + bg_ref[...]
    hlin = jnp.dot(hg, wf_ref[...], preferred_element_type=jnp.float32) + bf_ref[...]
    mu = jnp.mean(hlin, axis=1, keepdims=True)
    var = jnp.mean((hlin - mu) ** 2, axis=1, keepdims=True)
    out_ref[...] = (hlin - mu) * lax.rsqrt(var + 1e-5) * g_ref[...] + b_ref[...]


def _tc3(num, sl, xl, bias_gat, W_fin, b_fin, ln_gamma, ln_beta):
    BN = 400
    return pl.pallas_call(
        _tc3_body,
        grid=(N // BN,),
        in_specs=[
            pl.BlockSpec((BN, XW), lambda i: (i, 0)),
            pl.BlockSpec((BN, 16), lambda i: (i, 0)),
            pl.BlockSpec((BN, XW), lambda i: (i, 0)),
            pl.BlockSpec((1, HF), lambda i: (0, 0)),
            pl.BlockSpec((HF, F_OUT), lambda i: (0, 0)),
            pl.BlockSpec((1, F_OUT), lambda i: (0, 0)),
            pl.BlockSpec((1, F_OUT), lambda i: (0, 0)),
            pl.BlockSpec((1, F_OUT), lambda i: (0, 0)),
        ],
        out_specs=pl.BlockSpec((BN, F_OUT), lambda i: (i, 0)),
        out_shape=jax.ShapeDtypeStruct((N, F_OUT), jnp.float32),
    )(num, sl, xl, bias_gat.reshape(1, HF), W_fin, b_fin.reshape(1, F_OUT),
      ln_gamma.reshape(1, F_OUT), ln_beta.reshape(1, F_OUT))


def kernel(X, E_idx, attr, W_l, b_l, W_r, b_r, att, bias_gat, W_fin, b_fin,
           ln_gamma, ln_beta):
    att_flat = att.reshape(HF)
    src = E_idx[0]
    dst = E_idx[1]
    xl, xr, sl = _tc1(X, W_l, b_l, W_r, b_r, att_flat)
    part = _sca(xl, xr, src, dst, att_flat)
    expv = _tc2(part)
    num = _scb(xl, expv.reshape(E * H), src, dst)
    h = _tc3(num[:N], sl, xl, bias_gat, W_fin, b_fin, ln_gamma, ln_beta)
    return (h, E_idx, attr)


# speedup vs baseline: 6.4344x; 1.0270x over previous
import functools

import jax
import jax.numpy as jnp
from jax import lax
from jax.experimental import pallas as pl
from jax.experimental.pallas import tpu as pltpu
from jax.experimental.pallas import tpu_sc as plsc

N = 10000
E = 160000
F_IN = 256
F_OUT = 256
H = 4
HF = H * F_OUT
XW = HF + 128
NEG = 0.2

NC = 2
NS = 16
NW = NC * NS

EPW = E // NW
GA = 8
T_ROWS = 64
ACC_ROWS = 72
TRASH = 66
SB = 1600
GB = 32
N_SLICES = 5
N_PAD = N_SLICES * NW * T_ROWS


def _tc1_body(x_ref, wl_ref, bl_ref, wr_ref, br_ref, att_ref,
              xl_ref, xr_ref, sl_ref):
    x = x_ref[...]
    xl = jnp.dot(x, wl_ref[...], preferred_element_type=jnp.float32) + bl_ref[...]
    xr = jnp.dot(x, wr_ref[...], preferred_element_type=jnp.float32) + br_ref[...]
    bn = x.shape[0]
    ci = lax.broadcasted_iota(jnp.int32, (bn, XW - HF), 1)
    aug = jnp.where((ci % 16 == 0) & (ci < 16 * H), 1.0, 0.0).astype(jnp.float32)
    xl_ref[...] = jnp.concatenate([xl, aug], axis=1)
    xr_ref[...] = xr
    z = jnp.maximum(xl + xr, NEG * (xl + xr))
    cols = []
    for h in range(H):
        zh = z[:, h * F_OUT:(h + 1) * F_OUT] * att_ref[:, h * F_OUT:(h + 1) * F_OUT]
        cols.append(jnp.sum(zh, axis=1, keepdims=True))
    cols.append(jnp.zeros((bn, 16 - H), jnp.float32))
    sl_ref[...] = jnp.concatenate(cols, axis=1)


def _tc1(X, W_l, b_l, W_r, b_r, att_flat):
    BN = 400
    return pl.pallas_call(
        _tc1_body,
        grid=(N // BN,),
        in_specs=[
            pl.BlockSpec((BN, F_IN), lambda i: (i, 0)),
            pl.BlockSpec((F_IN, HF), lambda i: (0, 0)),
            pl.BlockSpec((1, HF), lambda i: (0, 0)),
            pl.BlockSpec((F_IN, HF), lambda i: (0, 0)),
            pl.BlockSpec((1, HF), lambda i: (0, 0)),
            pl.BlockSpec((1, HF), lambda i: (0, 0)),
        ],
        out_specs=[
            pl.BlockSpec((BN, XW), lambda i: (i, 0)),
            pl.BlockSpec((BN, HF), lambda i: (i, 0)),
            pl.BlockSpec((BN, 16), lambda i: (i, 0)),
        ],
        out_shape=[
            jax.ShapeDtypeStruct((N, XW), jnp.float32),
            jax.ShapeDtypeStruct((N, HF), jnp.float32),
            jax.ShapeDtypeStruct((N, 16), jnp.float32),
        ],
    )(X, W_l, b_l.reshape(1, HF), W_r, b_r.reshape(1, HF), att_flat.reshape(1, HF))


def _sca_body(xl_hbm, xr_hbm, src_hbm, dst_hbm, att_hbm,
              part_hbm,
              src_v, dst_v, att_v, xlr0_v, xlr1_v, xrr0_v, xrr1_v,
              out0_v, out1_v, semg0, semg1, semo0, semo1):
    wid = lax.axis_index("s") * NC + lax.axis_index("c")
    base = wid * EPW
    NCH = EPW // GA
    pltpu.sync_copy(src_hbm.at[pl.ds(base, EPW)], src_v)
    pltpu.sync_copy(dst_hbm.at[pl.ds(base, EPW)], dst_v)
    pltpu.sync_copy(att_hbm, att_v)

    def fire(g, xlr, xrr, semg):
        eb = g * GA
        pltpu.async_copy(xl_hbm.at[src_v.at[pl.ds(eb, GA)]], xlr, semg)
        pltpu.async_copy(xr_hbm.at[dst_v.at[pl.ds(eb, GA)]], xrr, semg)

    def half(g, xlr, xrr, out_v, semg, semo):
        eb = g * GA
        pltpu.make_async_copy(xl_hbm.at[src_v.at[pl.ds(eb, GA)]], xlr,
                              semg).wait()
        pltpu.make_async_copy(xr_hbm.at[dst_v.at[pl.ds(eb, GA)]], xrr,
                              semg).wait()

        @pl.when(g >= 2)
        def _():
            pltpu.make_async_copy(out_v, part_hbm.at[pl.ds(base, GA)],
                                  semo).wait()

        def edge(e, _):
            for h in range(H):
                acc = jnp.zeros((16,), jnp.float32)
                for j in range(16):
                    off = h * F_OUT + j * 16
                    z = xlr[e, pl.ds(off, 16)] + xrr[e, pl.ds(off, 16)]
                    acc = acc + jnp.maximum(z, NEG * z) * att_v[pl.ds(off, 16)]
                out_v[e, pl.ds(h * 16, 16)] = acc
            return 0
        lax.fori_loop(0, GA, edge, 0)

        pltpu.async_copy(out_v, part_hbm.at[pl.ds(base + eb, GA)], semo)

        @pl.when(g + 2 < NCH)
        def _():
            fire(g + 2, xlr, xrr, semg)

    fire(0, xlr0_v, xrr0_v, semg0)
    fire(1, xlr1_v, xrr1_v, semg1)

    def chunk(g, _):
        @pl.when(g % 2 == 0)
        def _():
            half(g, xlr0_v, xrr0_v, out0_v, semg0, semo0)
        @pl.when(g % 2 == 1)
        def _():
            half(g, xlr1_v, xrr1_v, out1_v, semg1, semo1)
        return 0
    lax.fori_loop(0, NCH, chunk, 0)

    pltpu.make_async_copy(out0_v, part_hbm.at[pl.ds(base, GA)], semo0).wait()
    pltpu.make_async_copy(out1_v, part_hbm.at[pl.ds(base, GA)], semo1).wait()


def _sca(xl, xr, src, dst, att_flat):
    mesh = plsc.VectorSubcoreMesh(core_axis_name="c", subcore_axis_name="s")
    f = functools.partial(
        pl.kernel,
        mesh=mesh,
        compiler_params=pltpu.CompilerParams(needs_layout_passes=False),
        out_type=jax.ShapeDtypeStruct((E, 64), jnp.float32),
        scratch_types=[
            pltpu.VMEM((EPW,), jnp.int32),
            pltpu.VMEM((EPW,), jnp.int32),
            pltpu.VMEM((HF,), jnp.float32),
            pltpu.VMEM((GA, XW), jnp.float32),
            pltpu.VMEM((GA, XW), jnp.float32),
            pltpu.VMEM((GA, HF), jnp.float32),
            pltpu.VMEM((GA, HF), jnp.float32),
            pltpu.VMEM((GA, 64), jnp.float32),
            pltpu.VMEM((GA, 64), jnp.float32),
            pltpu.SemaphoreType.DMA,
            pltpu.SemaphoreType.DMA,
            pltpu.SemaphoreType.DMA,
            pltpu.SemaphoreType.DMA,
        ],
    )(_sca_body)
    return f(xl, xr, src, dst, att_flat)


def _tc2_body(part_ref, expv_ref):
    p = part_ref[...]
    cols = []
    for h in range(H):
        l_h = jnp.sum(p[:, h * 16:(h + 1) * 16], axis=1, keepdims=True)
        cols.append(jnp.exp(jnp.minimum(l_h, 60.0)))
    expv_ref[...] = jnp.concatenate(cols, axis=1)


def _tc2(part):
    BE = 4000
    return pl.pallas_call(
        _tc2_body,
        grid=(E // BE,),
        in_specs=[pl.BlockSpec((BE, 64), lambda i: (i, 0))],
        out_specs=pl.BlockSpec((BE, H), lambda i: (i, 0)),
        out_shape=jax.ShapeDtypeStruct((E, H), jnp.float32),
    )(part)


def _scb_body(xl_hbm, expv_hbm, src_hbm, dst_hbm,
              num_hbm,
              dblk0_v, dblk1_v, sblk0_v, sblk1_v, pkl_v, srl_v, ldst_v,
              evidx_v, evgot_v, rows_v, acc_v, semd, seme):
    c = lax.axis_index("c")
    s = lax.axis_index("s")
    wid = s * NC + c
    zeros16 = jnp.zeros((16,), jnp.float32)
    iota = lax.iota(jnp.int32, 16)
    ldst_v[pl.ds(GB, 16)] = jnp.full((16,), TRASH, jnp.int32)
    NB = E // SB

    def do_pass(p, _):
        lo = p * (NW * T_ROWS) + wid * T_ROWS
        for q in range(GB // 16 + 1):
            srl_v[pl.ds(q * 16, 16)] = jnp.zeros((16,), jnp.int32)

        def zacc(r, _):
            for j in range(XW // 16):
                acc_v[r, pl.ds(j * 16, 16)] = zeros16
            return 0
        lax.fori_loop(0, ACC_ROWS, zacc, 0)

        def do_chunk(base, avail):
            for q in range(GB // 16):
                pk = pkl_v[pl.ds(base + q * 16, 16)]
                gq = lax.shift_right_logical(pk, 6)
                ldq = lax.bitwise_and(pk, 63)
                vq = (q * 16 + iota) < avail
                gq = jnp.where(vq, gq, 0)
                ldst_v[pl.ds(q * 16, 16)] = jnp.where(vq, ldq, TRASH)
                for h in range(H):
                    plsc.store_scatter(
                        evidx_v.at[pl.ds(q * 64, 64)],
                        [iota * 4 + h], gq * 4 + h)
            pltpu.async_copy(expv_hbm.at[evidx_v], evgot_v.at[pl.ds(0, 4 * GB)],
                             seme)
            pltpu.sync_copy(xl_hbm.at[srl_v.at[pl.ds(base, GB)]], rows_v)
            pltpu.make_async_copy(expv_hbm.at[evidx_v],
                                  evgot_v.at[pl.ds(0, 4 * GB)], seme).wait()

            def edge(e, _):
                evrow = evgot_v[pl.ds(e * 4, 16)]
                ld = ldst_v[pl.ds(e, 16)][0]
                for h in range(H):
                    evs = evrow[h]
                    for j in range(16):
                        off = h * F_OUT + j * 16
                        plsc.addupdate(acc_v.at[ld, pl.ds(off, 16)],
                                       rows_v[e, pl.ds(off, 16)] * evs)
                    offz = HF + h * 16
                    plsc.addupdate(acc_v.at[ld, pl.ds(offz, 16)],
                                   rows_v[e, pl.ds(offz, 16)] * evs)
                return 0
            lax.fori_loop(0, GB, edge, 0)

        pltpu.async_copy(dst_hbm.at[pl.ds(0, SB)], dblk0_v, semd)
        pltpu.async_copy(src_hbm.at[pl.ds(0, SB)], sblk0_v, semd)

        def blk(b, rem):
            def scan_into(dblk_v, sblk_v):
                pltpu.make_async_copy(dst_hbm.at[pl.ds(b * SB, SB)], dblk_v,
                                      semd).wait()
                pltpu.make_async_copy(src_hbm.at[pl.ds(b * SB, SB)], sblk_v,
                                      semd).wait()

                def scan(i, cnt):
                    d16 = dblk_v[pl.ds(i * 16, 16)]
                    s16 = sblk_v[pl.ds(i * 16, 16)]
                    m = (d16 >= lo) & (d16 < lo + T_ROWS)
                    pk = (b * SB + i * 16 + iota) * 64 + (d16 - lo)
                    plsc.store_compressed(pkl_v.at[pl.ds(cnt, 16)], pk,
                                          mask=m)
                    plsc.store_compressed(srl_v.at[pl.ds(cnt, 16)], s16,
                                          mask=m)
                    pc = plsc.all_reduce_population_count(m)
                    return cnt + pc[0]
                return lax.fori_loop(0, SB // 16, scan, rem)

            @pl.when(b % 2 == 0)
            def _():
                @pl.when(b + 1 < NB)
                def _():
                    pltpu.async_copy(dst_hbm.at[pl.ds((b + 1) * SB, SB)],
                                     dblk1_v, semd)
                    pltpu.async_copy(src_hbm.at[pl.ds((b + 1) * SB, SB)],
                                     sblk1_v, semd)
            @pl.when(b % 2 == 1)
            def _():
                @pl.when(b + 1 < NB)
                def _():
                    pltpu.async_copy(dst_hbm.at[pl.ds((b + 1) * SB, SB)],
                                     dblk0_v, semd)
                    pltpu.async_copy(src_hbm.at[pl.ds((b + 1) * SB, SB)],
                                     sblk0_v, semd)
            avail0 = lax.cond(b % 2 == 0,
                              lambda: scan_into(dblk0_v, sblk0_v),
                              lambda: scan_into(dblk1_v, sblk1_v))

            nfull = lax.div(avail0, GB)

            def chunk(k, _):
                do_chunk(k * GB, GB)
                return 0
            lax.fori_loop(0, nfull, chunk, 0)

            rem2 = avail0 - nfull * GB
            tb = nfull * GB
            for q in range(GB // 16):
                pq = pkl_v[pl.ds(tb + q * 16, 16)]
                sq = srl_v[pl.ds(tb + q * 16, 16)]
                pkl_v[pl.ds(q * 16, 16)] = pq
                srl_v[pl.ds(q * 16, 16)] = sq
            return rem2

        rem = lax.fori_loop(0, NB, blk, 0)

        @pl.when(rem > 0)
        def _():
            do_chunk(0, rem)

        pltpu.sync_copy(acc_v.at[pl.ds(0, T_ROWS)],
                        num_hbm.at[pl.ds(lo, T_ROWS)])
        return 0

    lax.fori_loop(0, N_SLICES, do_pass, 0)


def _scb(xl, expv_flat, src, dst):
    mesh = plsc.VectorSubcoreMesh(core_axis_name="c", subcore_axis_name="s")
    f = functools.partial(
        pl.kernel,
        mesh=mesh,
        compiler_params=pltpu.CompilerParams(needs_layout_passes=False),
        out_type=jax.ShapeDtypeStruct((N_PAD, XW), jnp.float32),
        scratch_types=[
            pltpu.VMEM((SB,), jnp.int32),
            pltpu.VMEM((SB,), jnp.int32),
            pltpu.VMEM((SB,), jnp.int32),
            pltpu.VMEM((SB,), jnp.int32),
            pltpu.VMEM((SB + GB + 32,), jnp.int32),
            pltpu.VMEM((SB + GB + 32,), jnp.int32),
            pltpu.VMEM((GB + 16,), jnp.int32),
            pltpu.VMEM((4 * GB,), jnp.int32),
            pltpu.VMEM((4 * GB + 16,), jnp.float32),
            pltpu.VMEM((GB, XW), jnp.float32),
            pltpu.VMEM((ACC_ROWS, XW), jnp.float32),
            pltpu.SemaphoreType.DMA,
            pltpu.SemaphoreType.DMA,
        ],
    )(_scb_body)
    return f(xl, expv_flat, src, dst)


def _tc3_body(num_ref, sl_ref, xl_ref, bg_ref, wf_ref, bf_ref, g_ref,
              b_ref, out_ref):
    cols = []
    for h in range(H):
        es = jnp.exp(jnp.minimum(sl_ref[:, h:h + 1], 60.0))
        znorm = num_ref[:, HF + 16 * h:HF + 16 * h + 1] + es + 1e-16
        seg = (num_ref[:, h * F_OUT:(h + 1) * F_OUT]
               + es * xl_ref[:, h * F_OUT:(h + 1) * F_OUT]) / znorm
        cols.append(seg)
    hg = jnp.concatenate(cols, axis=1) + bg_ref[...]
    hlin = jnp.dot(hg, wf_ref[...], preferred_element_type=jnp.float32) + bf_ref[...]
    mu = jnp.mean(hlin, axis=1, keepdims=True)
    var = jnp.mean((hlin - mu) ** 2, axis=1, keepdims=True)
    out_ref[...] = (hlin - mu) * lax.rsqrt(var + 1e-5) * g_ref[...] + b_ref[...]


def _tc3(num, sl, xl, bias_gat, W_fin, b_fin, ln_gamma, ln_beta):
    BN = 400
    return pl.pallas_call(
        _tc3_body,
        grid=(N // BN,),
        in_specs=[
            pl.BlockSpec((BN, XW), lambda i: (i, 0)),
            pl.BlockSpec((BN, 16), lambda i: (i, 0)),
            pl.BlockSpec((BN, XW), lambda i: (i, 0)),
            pl.BlockSpec((1, HF), lambda i: (0, 0)),
            pl.BlockSpec((HF, F_OUT), lambda i: (0, 0)),
            pl.BlockSpec((1, F_OUT), lambda i: (0, 0)),
            pl.BlockSpec((1, F_OUT), lambda i: (0, 0)),
            pl.BlockSpec((1, F_OUT), lambda i: (0, 0)),
        ],
        out_specs=pl.BlockSpec((BN, F_OUT), lambda i: (i, 0)),
        out_shape=jax.ShapeDtypeStruct((N, F_OUT), jnp.float32),
    )(num, sl, xl, bias_gat.reshape(1, HF), W_fin, b_fin.reshape(1, F_OUT),
      ln_gamma.reshape(1, F_OUT), ln_beta.reshape(1, F_OUT))


def kernel(X, E_idx, attr, W_l, b_l, W_r, b_r, att, bias_gat, W_fin, b_fin,
           ln_gamma, ln_beta):
    att_flat = att.reshape(HF)
    src = E_idx[0]
    dst = E_idx[1]
    xl, xr, sl = _tc1(X, W_l, b_l, W_r, b_r, att_flat)
    part = _sca(xl, xr, src, dst, att_flat)
    expv = _tc2(part)
    num = _scb(xl, expv.reshape(E * H), src, dst)
    h = _tc3(num[:N], sl, xl, bias_gat, W_fin, b_fin, ln_gamma, ln_beta)
    return (h, E_idx, attr)


# speedup vs baseline: 6.9644x; 1.0824x over previous
import functools

import jax
import jax.numpy as jnp
from jax import lax
from jax.experimental import pallas as pl
from jax.experimental.pallas import tpu as pltpu
from jax.experimental.pallas import tpu_sc as plsc

N = 10000
E = 160000
F_IN = 256
F_OUT = 256
H = 4
HF = H * F_OUT
XW = HF + 128
NEG = 0.2

NC = 2
NS = 16
NW = NC * NS

EPW = E // NW
GA = 8
T_ROWS = 64
ACC_ROWS = 72
TRASH = 66
SB = 1600
GB = 32
N_SLICES = 5
N_PAD = N_SLICES * NW * T_ROWS


def _tc1_body(x_ref, wl_ref, bl_ref, wr_ref, br_ref, att_ref,
              xl_ref, xr_ref, sl_ref):
    x = x_ref[...]
    xl = jnp.dot(x, wl_ref[...], preferred_element_type=jnp.float32) + bl_ref[...]
    xr = jnp.dot(x, wr_ref[...], preferred_element_type=jnp.float32) + br_ref[...]
    bn = x.shape[0]
    ci = lax.broadcasted_iota(jnp.int32, (bn, XW - HF), 1)
    aug = jnp.where((ci % 16 == 0) & (ci < 16 * H), 1.0, 0.0).astype(jnp.float32)
    xl_ref[...] = jnp.concatenate([xl, aug], axis=1)
    xr_ref[...] = xr
    z = jnp.maximum(xl + xr, NEG * (xl + xr))
    cols = []
    for h in range(H):
        zh = z[:, h * F_OUT:(h + 1) * F_OUT] * att_ref[:, h * F_OUT:(h + 1) * F_OUT]
        cols.append(jnp.sum(zh, axis=1, keepdims=True))
    cols.append(jnp.zeros((bn, 16 - H), jnp.float32))
    sl_ref[...] = jnp.concatenate(cols, axis=1)


def _tc1(X, W_l, b_l, W_r, b_r, att_flat):
    BN = 400
    return pl.pallas_call(
        _tc1_body,
        grid=(N // BN,),
        in_specs=[
            pl.BlockSpec((BN, F_IN), lambda i: (i, 0)),
            pl.BlockSpec((F_IN, HF), lambda i: (0, 0)),
            pl.BlockSpec((1, HF), lambda i: (0, 0)),
            pl.BlockSpec((F_IN, HF), lambda i: (0, 0)),
            pl.BlockSpec((1, HF), lambda i: (0, 0)),
            pl.BlockSpec((1, HF), lambda i: (0, 0)),
        ],
        out_specs=[
            pl.BlockSpec((BN, XW), lambda i: (i, 0)),
            pl.BlockSpec((BN, HF), lambda i: (i, 0)),
            pl.BlockSpec((BN, 16), lambda i: (i, 0)),
        ],
        out_shape=[
            jax.ShapeDtypeStruct((N, XW), jnp.float32),
            jax.ShapeDtypeStruct((N, HF), jnp.float32),
            jax.ShapeDtypeStruct((N, 16), jnp.float32),
        ],
    )(X, W_l, b_l.reshape(1, HF), W_r, b_r.reshape(1, HF), att_flat.reshape(1, HF))


def _sca_body(xl_hbm, xr_hbm, src_hbm, dst_hbm, att_hbm,
              part_hbm,
              src_v, dst_v, att_v, xlr0_v, xlr1_v, xrr0_v, xrr1_v,
              out0_v, out1_v, semg0, semg1, semo0, semo1):
    wid = lax.axis_index("s") * NC + lax.axis_index("c")
    base = wid * EPW
    NCH = EPW // GA
    pltpu.sync_copy(src_hbm.at[pl.ds(base, EPW)], src_v)
    pltpu.sync_copy(dst_hbm.at[pl.ds(base, EPW)], dst_v)
    pltpu.sync_copy(att_hbm, att_v)

    def fire(g, xlr, xrr, semg):
        eb = g * GA
        pltpu.async_copy(xl_hbm.at[src_v.at[pl.ds(eb, GA)]], xlr, semg)
        pltpu.async_copy(xr_hbm.at[dst_v.at[pl.ds(eb, GA)]], xrr, semg)

    def half(g, xlr, xrr, out_v, semg, semo):
        eb = g * GA
        pltpu.make_async_copy(xl_hbm.at[src_v.at[pl.ds(eb, GA)]], xlr,
                              semg).wait()
        pltpu.make_async_copy(xr_hbm.at[dst_v.at[pl.ds(eb, GA)]], xrr,
                              semg).wait()

        @pl.when(g >= 2)
        def _():
            pltpu.make_async_copy(out_v, part_hbm.at[pl.ds(base, GA)],
                                  semo).wait()

        def edge(e, _):
            for h in range(H):
                acc = jnp.zeros((16,), jnp.float32)
                for j in range(16):
                    off = h * F_OUT + j * 16
                    z = xlr[e, pl.ds(off, 16)] + xrr[e, pl.ds(off, 16)]
                    acc = acc + jnp.maximum(z, NEG * z) * att_v[pl.ds(off, 16)]
                out_v[e, pl.ds(h * 16, 16)] = acc
            return 0
        lax.fori_loop(0, GA, edge, 0)

        pltpu.async_copy(out_v, part_hbm.at[pl.ds(base + eb, GA)], semo)

        @pl.when(g + 2 < NCH)
        def _():
            fire(g + 2, xlr, xrr, semg)

    fire(0, xlr0_v, xrr0_v, semg0)
    fire(1, xlr1_v, xrr1_v, semg1)

    def chunk(g, _):
        @pl.when(g % 2 == 0)
        def _():
            half(g, xlr0_v, xrr0_v, out0_v, semg0, semo0)
        @pl.when(g % 2 == 1)
        def _():
            half(g, xlr1_v, xrr1_v, out1_v, semg1, semo1)
        return 0
    lax.fori_loop(0, NCH, chunk, 0)

    pltpu.make_async_copy(out0_v, part_hbm.at[pl.ds(base, GA)], semo0).wait()
    pltpu.make_async_copy(out1_v, part_hbm.at[pl.ds(base, GA)], semo1).wait()


def _sca(xl, xr, src, dst, att_flat):
    mesh = plsc.VectorSubcoreMesh(core_axis_name="c", subcore_axis_name="s")
    f = functools.partial(
        pl.kernel,
        mesh=mesh,
        compiler_params=pltpu.CompilerParams(needs_layout_passes=False),
        out_type=jax.ShapeDtypeStruct((E, 64), jnp.float32),
        scratch_types=[
            pltpu.VMEM((EPW,), jnp.int32),
            pltpu.VMEM((EPW,), jnp.int32),
            pltpu.VMEM((HF,), jnp.float32),
            pltpu.VMEM((GA, XW), jnp.float32),
            pltpu.VMEM((GA, XW), jnp.float32),
            pltpu.VMEM((GA, HF), jnp.float32),
            pltpu.VMEM((GA, HF), jnp.float32),
            pltpu.VMEM((GA, 64), jnp.float32),
            pltpu.VMEM((GA, 64), jnp.float32),
            pltpu.SemaphoreType.DMA,
            pltpu.SemaphoreType.DMA,
            pltpu.SemaphoreType.DMA,
            pltpu.SemaphoreType.DMA,
        ],
    )(_sca_body)
    return f(xl, xr, src, dst, att_flat)


def _tc2_body(part_ref, expv_ref):
    p = part_ref[...]
    cols = []
    for h in range(H):
        l_h = jnp.sum(p[:, h * 16:(h + 1) * 16], axis=1, keepdims=True)
        cols.append(jnp.exp(jnp.minimum(l_h, 60.0)))
    expv_ref[...] = jnp.concatenate(cols, axis=1)


def _tc2(part):
    BE = 4000
    return pl.pallas_call(
        _tc2_body,
        grid=(E // BE,),
        in_specs=[pl.BlockSpec((BE, 64), lambda i: (i, 0))],
        out_specs=pl.BlockSpec((BE, H), lambda i: (i, 0)),
        out_shape=jax.ShapeDtypeStruct((E, H), jnp.float32),
    )(part)


def _scb_body(xl_hbm, expv_hbm, src_hbm, dst_hbm,
              num_hbm,
              dblk0_v, dblk1_v, sblk0_v, sblk1_v, pkl_v, srl_v, ldst_v,
              evidx_v, evgot_v, rows_v, acc_v, semd, seme):
    c = lax.axis_index("c")
    s = lax.axis_index("s")
    wid = s * NC + c
    zeros16 = jnp.zeros((16,), jnp.float32)
    iota = lax.iota(jnp.int32, 16)
    ldst_v[pl.ds(GB, 16)] = jnp.full((16,), TRASH, jnp.int32)
    NB = E // SB

    def do_pass(p, _):
        lo = p * (NW * T_ROWS) + wid * T_ROWS
        for q in range(GB // 16 + 1):
            srl_v[pl.ds(q * 16, 16)] = jnp.zeros((16,), jnp.int32)

        def zacc(r, _):
            for j in range(XW // 16):
                acc_v[r, pl.ds(j * 16, 16)] = zeros16
            return 0
        lax.fori_loop(0, ACC_ROWS, zacc, 0)

        def do_chunk(base, avail):
            for q in range(GB // 16):
                pk = pkl_v[pl.ds(base + q * 16, 16)]
                gq = lax.shift_right_logical(pk, 6)
                ldq = lax.bitwise_and(pk, 63)
                vq = (q * 16 + iota) < avail
                gq = jnp.where(vq, gq, 0)
                ldst_v[pl.ds(q * 16, 16)] = jnp.where(vq, ldq, TRASH)
                for h in range(H):
                    plsc.store_scatter(
                        evidx_v.at[pl.ds(q * 64, 64)],
                        [iota * 4 + h], gq * 4 + h)
            pltpu.async_copy(expv_hbm.at[evidx_v], evgot_v.at[pl.ds(0, 4 * GB)],
                             seme)
            pltpu.sync_copy(xl_hbm.at[srl_v.at[pl.ds(base, GB)]], rows_v)
            pltpu.make_async_copy(expv_hbm.at[evidx_v],
                                  evgot_v.at[pl.ds(0, 4 * GB)], seme).wait()

            def edge(e, _):
                evrow = evgot_v[pl.ds(e * 4, 16)]
                ld = ldst_v[pl.ds(e, 16)][0]
                for h in range(H):
                    evs = evrow[h]
                    for j in range(16):
                        off = h * F_OUT + j * 16
                        plsc.addupdate(acc_v.at[ld, pl.ds(off, 16)],
                                       rows_v[e, pl.ds(off, 16)] * evs)
                    offz = HF + h * 16
                    plsc.addupdate(acc_v.at[ld, pl.ds(offz, 16)],
                                   rows_v[e, pl.ds(offz, 16)] * evs)
                return 0
            lax.fori_loop(0, GB, edge, 0)

        pltpu.async_copy(dst_hbm.at[pl.ds(0, SB)], dblk0_v, semd)
        pltpu.async_copy(src_hbm.at[pl.ds(0, SB)], sblk0_v, semd)

        def blk(b, rem):
            def scan_into(dblk_v, sblk_v):
                pltpu.make_async_copy(dst_hbm.at[pl.ds(b * SB, SB)], dblk_v,
                                      semd).wait()
                pltpu.make_async_copy(src_hbm.at[pl.ds(b * SB, SB)], sblk_v,
                                      semd).wait()

                def scan(i, cnt):
                    ms, pks, ss, pcs = [], [], [], []
                    for u in range(4):
                        d16 = dblk_v[pl.ds((i * 4 + u) * 16, 16)]
                        s16 = sblk_v[pl.ds((i * 4 + u) * 16, 16)]
                        m = (d16 >= lo) & (d16 < lo + T_ROWS)
                        pk = (b * SB + (i * 4 + u) * 16 + iota) * 64 + (d16 - lo)
                        ms.append(m)
                        pks.append(pk)
                        ss.append(s16)
                        pcs.append(plsc.all_reduce_population_count(m))
                    off = cnt
                    for u in range(4):
                        plsc.store_compressed(pkl_v.at[pl.ds(off, 16)],
                                              pks[u], mask=ms[u])
                        plsc.store_compressed(srl_v.at[pl.ds(off, 16)],
                                              ss[u], mask=ms[u])
                        if u < 3:
                            off = off + pcs[u][0]
                    total = pcs[0] + pcs[1] + pcs[2] + pcs[3]
                    return cnt + total[0]
                return lax.fori_loop(0, SB // 64, scan, rem)

            @pl.when(b % 2 == 0)
            def _():
                @pl.when(b + 1 < NB)
                def _():
                    pltpu.async_copy(dst_hbm.at[pl.ds((b + 1) * SB, SB)],
                                     dblk1_v, semd)
                    pltpu.async_copy(src_hbm.at[pl.ds((b + 1) * SB, SB)],
                                     sblk1_v, semd)
            @pl.when(b % 2 == 1)
            def _():
                @pl.when(b + 1 < NB)
                def _():
                    pltpu.async_copy(dst_hbm.at[pl.ds((b + 1) * SB, SB)],
                                     dblk0_v, semd)
                    pltpu.async_copy(src_hbm.at[pl.ds((b + 1) * SB, SB)],
                                     sblk0_v, semd)
            avail0 = lax.cond(b % 2 == 0,
                              lambda: scan_into(dblk0_v, sblk0_v),
                              lambda: scan_into(dblk1_v, sblk1_v))

            nfull = lax.div(avail0, GB)

            def chunk(k, _):
                do_chunk(k * GB, GB)
                return 0
            lax.fori_loop(0, nfull, chunk, 0)

            rem2 = avail0 - nfull * GB
            tb = nfull * GB
            for q in range(GB // 16):
                pq = pkl_v[pl.ds(tb + q * 16, 16)]
                sq = srl_v[pl.ds(tb + q * 16, 16)]
                pkl_v[pl.ds(q * 16, 16)] = pq
                srl_v[pl.ds(q * 16, 16)] = sq
            return rem2

        rem = lax.fori_loop(0, NB, blk, 0)

        @pl.when(rem > 0)
        def _():
            do_chunk(0, rem)

        pltpu.sync_copy(acc_v.at[pl.ds(0, T_ROWS)],
                        num_hbm.at[pl.ds(lo, T_ROWS)])
        return 0

    lax.fori_loop(0, N_SLICES, do_pass, 0)


def _scb(xl, expv_flat, src, dst):
    mesh = plsc.VectorSubcoreMesh(core_axis_name="c", subcore_axis_name="s")
    f = functools.partial(
        pl.kernel,
        mesh=mesh,
        compiler_params=pltpu.CompilerParams(needs_layout_passes=False),
        out_type=jax.ShapeDtypeStruct((N_PAD, XW), jnp.float32),
        scratch_types=[
            pltpu.VMEM((SB,), jnp.int32),
            pltpu.VMEM((SB,), jnp.int32),
            pltpu.VMEM((SB,), jnp.int32),
            pltpu.VMEM((SB,), jnp.int32),
            pltpu.VMEM((SB + GB + 32,), jnp.int32),
            pltpu.VMEM((SB + GB + 32,), jnp.int32),
            pltpu.VMEM((GB + 16,), jnp.int32),
            pltpu.VMEM((4 * GB,), jnp.int32),
            pltpu.VMEM((4 * GB + 16,), jnp.float32),
            pltpu.VMEM((GB, XW), jnp.float32),
            pltpu.VMEM((ACC_ROWS, XW), jnp.float32),
            pltpu.SemaphoreType.DMA,
            pltpu.SemaphoreType.DMA,
        ],
    )(_scb_body)
    return f(xl, expv_flat, src, dst)


def _tc3_body(num_ref, sl_ref, xl_ref, bg_ref, wf_ref, bf_ref, g_ref,
              b_ref, out_ref):
    cols = []
    for h in range(H):
        es = jnp.exp(jnp.minimum(sl_ref[:, h:h + 1], 60.0))
        znorm = num_ref[:, HF + 16 * h:HF + 16 * h + 1] + es + 1e-16
        seg = (num_ref[:, h * F_OUT:(h + 1) * F_OUT]
               + es * xl_ref[:, h * F_OUT:(h + 1) * F_OUT]) / znorm
        cols.append(seg)
    hg = jnp.concatenate(cols, axis=1) + bg_ref[...]
    hlin = jnp.dot(hg, wf_ref[...], preferred_element_type=jnp.float32) + bf_ref[...]
    mu = jnp.mean(hlin, axis=1, keepdims=True)
    var = jnp.mean((hlin - mu) ** 2, axis=1, keepdims=True)
    out_ref[...] = (hlin - mu) * lax.rsqrt(var + 1e-5) * g_ref[...] + b_ref[...]


def _tc3(num, sl, xl, bias_gat, W_fin, b_fin, ln_gamma, ln_beta):
    BN = 400
    return pl.pallas_call(
        _tc3_body,
        grid=(N // BN,),
        in_specs=[
            pl.BlockSpec((BN, XW), lambda i: (i, 0)),
            pl.BlockSpec((BN, 16), lambda i: (i, 0)),
            pl.BlockSpec((BN, XW), lambda i: (i, 0)),
            pl.BlockSpec((1, HF), lambda i: (0, 0)),
            pl.BlockSpec((HF, F_OUT), lambda i: (0, 0)),
            pl.BlockSpec((1, F_OUT), lambda i: (0, 0)),
            pl.BlockSpec((1, F_OUT), lambda i: (0, 0)),
            pl.BlockSpec((1, F_OUT), lambda i: (0, 0)),
        ],
        out_specs=pl.BlockSpec((BN, F_OUT), lambda i: (i, 0)),
        out_shape=jax.ShapeDtypeStruct((N, F_OUT), jnp.float32),
    )(num, sl, xl, bias_gat.reshape(1, HF), W_fin, b_fin.reshape(1, F_OUT),
      ln_gamma.reshape(1, F_OUT), ln_beta.reshape(1, F_OUT))


def kernel(X, E_idx, attr, W_l, b_l, W_r, b_r, att, bias_gat, W_fin, b_fin,
           ln_gamma, ln_beta):
    att_flat = att.reshape(HF)
    src = E_idx[0]
    dst = E_idx[1]
    xl, xr, sl = _tc1(X, W_l, b_l, W_r, b_r, att_flat)
    part = _sca(xl, xr, src, dst, att_flat)
    expv = _tc2(part)
    num = _scb(xl, expv.reshape(E * H), src, dst)
    h = _tc3(num[:N], sl, xl, bias_gat, W_fin, b_fin, ln_gamma, ln_beta)
    return (h, E_idx, attr)


# speedup vs baseline: 7.2101x; 1.0353x over previous
import functools

import jax
import jax.numpy as jnp
from jax import lax
from jax.experimental import pallas as pl
from jax.experimental.pallas import tpu as pltpu
from jax.experimental.pallas import tpu_sc as plsc

N = 10000
E = 160000
F_IN = 256
F_OUT = 256
H = 4
HF = H * F_OUT
XW = HF + 128
NEG = 0.2

NC = 2
NS = 16
NW = NC * NS

EPW = E // NW
GA = 8
T_ROWS = 64
ACC_ROWS = 72
TRASH = 66
SB = 1600
GB = 16
N_SLICES = 5
N_PAD = N_SLICES * NW * T_ROWS


def _tc1_body(x_ref, wl_ref, bl_ref, wr_ref, br_ref, att_ref,
              xl_ref, xr_ref, sl_ref):
    x = x_ref[...]
    xl = jnp.dot(x, wl_ref[...], preferred_element_type=jnp.float32) + bl_ref[...]
    xr = jnp.dot(x, wr_ref[...], preferred_element_type=jnp.float32) + br_ref[...]
    bn = x.shape[0]
    ci = lax.broadcasted_iota(jnp.int32, (bn, XW - HF), 1)
    aug = jnp.where((ci % 16 == 0) & (ci < 16 * H), 1.0, 0.0).astype(jnp.float32)
    xl_ref[...] = jnp.concatenate([xl, aug], axis=1)
    xr_ref[...] = xr
    z = jnp.maximum(xl + xr, NEG * (xl + xr))
    cols = []
    for h in range(H):
        zh = z[:, h * F_OUT:(h + 1) * F_OUT] * att_ref[:, h * F_OUT:(h + 1) * F_OUT]
        cols.append(jnp.sum(zh, axis=1, keepdims=True))
    cols.append(jnp.zeros((bn, 16 - H), jnp.float32))
    sl_ref[...] = jnp.concatenate(cols, axis=1)


def _tc1(X, W_l, b_l, W_r, b_r, att_flat):
    BN = 400
    return pl.pallas_call(
        _tc1_body,
        grid=(N // BN,),
        in_specs=[
            pl.BlockSpec((BN, F_IN), lambda i: (i, 0)),
            pl.BlockSpec((F_IN, HF), lambda i: (0, 0)),
            pl.BlockSpec((1, HF), lambda i: (0, 0)),
            pl.BlockSpec((F_IN, HF), lambda i: (0, 0)),
            pl.BlockSpec((1, HF), lambda i: (0, 0)),
            pl.BlockSpec((1, HF), lambda i: (0, 0)),
        ],
        out_specs=[
            pl.BlockSpec((BN, XW), lambda i: (i, 0)),
            pl.BlockSpec((BN, HF), lambda i: (i, 0)),
            pl.BlockSpec((BN, 16), lambda i: (i, 0)),
        ],
        out_shape=[
            jax.ShapeDtypeStruct((N, XW), jnp.float32),
            jax.ShapeDtypeStruct((N, HF), jnp.float32),
            jax.ShapeDtypeStruct((N, 16), jnp.float32),
        ],
    )(X, W_l, b_l.reshape(1, HF), W_r, b_r.reshape(1, HF), att_flat.reshape(1, HF))


def _sca_body(xl_hbm, xr_hbm, src_hbm, dst_hbm, att_hbm,
              part_hbm,
              src_v, dst_v, att_v, xlr0_v, xlr1_v, xrr0_v, xrr1_v,
              out0_v, out1_v, semg0, semg1, semo0, semo1):
    wid = lax.axis_index("s") * NC + lax.axis_index("c")
    base = wid * EPW
    NCH = EPW // GA
    pltpu.sync_copy(src_hbm.at[pl.ds(base, EPW)], src_v)
    pltpu.sync_copy(dst_hbm.at[pl.ds(base, EPW)], dst_v)
    pltpu.sync_copy(att_hbm, att_v)

    def fire(g, xlr, xrr, semg):
        eb = g * GA
        pltpu.async_copy(xl_hbm.at[src_v.at[pl.ds(eb, GA)]], xlr, semg)
        pltpu.async_copy(xr_hbm.at[dst_v.at[pl.ds(eb, GA)]], xrr, semg)

    def half(g, xlr, xrr, out_v, semg, semo):
        eb = g * GA
        pltpu.make_async_copy(xl_hbm.at[src_v.at[pl.ds(eb, GA)]], xlr,
                              semg).wait()
        pltpu.make_async_copy(xr_hbm.at[dst_v.at[pl.ds(eb, GA)]], xrr,
                              semg).wait()

        @pl.when(g >= 2)
        def _():
            pltpu.make_async_copy(out_v, part_hbm.at[pl.ds(base, GA)],
                                  semo).wait()

        def edge(e, _):
            for h in range(H):
                acc = jnp.zeros((16,), jnp.float32)
                for j in range(16):
                    off = h * F_OUT + j * 16
                    z = xlr[e, pl.ds(off, 16)] + xrr[e, pl.ds(off, 16)]
                    acc = acc + jnp.maximum(z, NEG * z) * att_v[pl.ds(off, 16)]
                out_v[e, pl.ds(h * 16, 16)] = acc
            return 0
        lax.fori_loop(0, GA, edge, 0)

        pltpu.async_copy(out_v, part_hbm.at[pl.ds(base + eb, GA)], semo)

        @pl.when(g + 2 < NCH)
        def _():
            fire(g + 2, xlr, xrr, semg)

    fire(0, xlr0_v, xrr0_v, semg0)
    fire(1, xlr1_v, xrr1_v, semg1)

    def chunk(g, _):
        @pl.when(g % 2 == 0)
        def _():
            half(g, xlr0_v, xrr0_v, out0_v, semg0, semo0)
        @pl.when(g % 2 == 1)
        def _():
            half(g, xlr1_v, xrr1_v, out1_v, semg1, semo1)
        return 0
    lax.fori_loop(0, NCH, chunk, 0)

    pltpu.make_async_copy(out0_v, part_hbm.at[pl.ds(base, GA)], semo0).wait()
    pltpu.make_async_copy(out1_v, part_hbm.at[pl.ds(base, GA)], semo1).wait()


def _sca(xl, xr, src, dst, att_flat):
    mesh = plsc.VectorSubcoreMesh(core_axis_name="c", subcore_axis_name="s")
    f = functools.partial(
        pl.kernel,
        mesh=mesh,
        compiler_params=pltpu.CompilerParams(needs_layout_passes=False),
        out_type=jax.ShapeDtypeStruct((E, 64), jnp.float32),
        scratch_types=[
            pltpu.VMEM((EPW,), jnp.int32),
            pltpu.VMEM((EPW,), jnp.int32),
            pltpu.VMEM((HF,), jnp.float32),
            pltpu.VMEM((GA, XW), jnp.float32),
            pltpu.VMEM((GA, XW), jnp.float32),
            pltpu.VMEM((GA, HF), jnp.float32),
            pltpu.VMEM((GA, HF), jnp.float32),
            pltpu.VMEM((GA, 64), jnp.float32),
            pltpu.VMEM((GA, 64), jnp.float32),
            pltpu.SemaphoreType.DMA,
            pltpu.SemaphoreType.DMA,
            pltpu.SemaphoreType.DMA,
            pltpu.SemaphoreType.DMA,
        ],
    )(_sca_body)
    return f(xl, xr, src, dst, att_flat)


def _tc2_body(part_ref, expv_ref):
    p = part_ref[...]
    cols = []
    for h in range(H):
        l_h = jnp.sum(p[:, h * 16:(h + 1) * 16], axis=1, keepdims=True)
        cols.append(jnp.exp(jnp.minimum(l_h, 60.0)))
    expv_ref[...] = jnp.concatenate(cols, axis=1)


def _tc2(part):
    BE = 4000
    return pl.pallas_call(
        _tc2_body,
        grid=(E // BE,),
        in_specs=[pl.BlockSpec((BE, 64), lambda i: (i, 0))],
        out_specs=pl.BlockSpec((BE, H), lambda i: (i, 0)),
        out_shape=jax.ShapeDtypeStruct((E, H), jnp.float32),
    )(part)


def _scb_body(xl_hbm, expv_hbm, src_hbm, dst_hbm,
              num_hbm,
              dblk0_v, dblk1_v, sblk0_v, sblk1_v, pkl_v, srl_v,
              lds0_v, lds1_v, sidx0_v, sidx1_v, evi0_v, evi1_v,
              evg0_v, evg1_v, rows0_v, rows1_v, acc_v,
              semd, semx0, semx1, seme0, seme1):
    c = lax.axis_index("c")
    s = lax.axis_index("s")
    wid = s * NC + c
    zeros16 = jnp.zeros((16,), jnp.float32)
    iota = lax.iota(jnp.int32, 16)
    lds0_v[pl.ds(GB, 16)] = jnp.full((16,), TRASH, jnp.int32)
    lds1_v[pl.ds(GB, 16)] = jnp.full((16,), TRASH, jnp.int32)
    NB = E // SB
    SLOT = [(lds0_v, sidx0_v, evi0_v, evg0_v, rows0_v, semx0, seme0),
            (lds1_v, sidx1_v, evi1_v, evg1_v, rows1_v, semx1, seme1)]

    def fire_s(sl, base, avail):
        lds_v, sidx_v, evi_v, evg_v, rows_v, semx, seme = SLOT[sl]
        pk = pkl_v[pl.ds(base, 16)]
        vq = iota < avail
        gq = jnp.where(vq, lax.shift_right_logical(pk, 6), 0)
        lds_v[pl.ds(0, 16)] = jnp.where(vq, lax.bitwise_and(pk, 63), TRASH)
        sidx_v[pl.ds(0, 16)] = srl_v[pl.ds(base, 16)]
        for h in range(H):
            plsc.store_scatter(evi_v, [iota * 4 + h], gq * 4 + h)
        pltpu.async_copy(expv_hbm.at[evi_v], evg_v.at[pl.ds(0, 4 * GB)], seme)
        pltpu.async_copy(xl_hbm.at[sidx_v], rows_v, semx)

    def comp_s(sl):
        lds_v, sidx_v, evi_v, evg_v, rows_v, semx, seme = SLOT[sl]
        pltpu.make_async_copy(xl_hbm.at[sidx_v], rows_v, semx).wait()
        pltpu.make_async_copy(expv_hbm.at[evi_v], evg_v.at[pl.ds(0, 4 * GB)],
                              seme).wait()

        def edge(e, _):
            evrow = evg_v[pl.ds(e * 4, 16)]
            ld = lds_v[pl.ds(e, 16)][0]
            for h in range(H):
                evs = evrow[h]
                for j in range(16):
                    off = h * F_OUT + j * 16
                    plsc.addupdate(acc_v.at[ld, pl.ds(off, 16)],
                                   rows_v[e, pl.ds(off, 16)] * evs)
                offz = HF + h * 16
                plsc.addupdate(acc_v.at[ld, pl.ds(offz, 16)],
                               rows_v[e, pl.ds(offz, 16)] * evs)
            return 0
        lax.fori_loop(0, GB, edge, 0)

    def fire_dyn(sl, base, avail):
        @pl.when(sl == 0)
        def _():
            fire_s(0, base, avail)
        @pl.when(sl == 1)
        def _():
            fire_s(1, base, avail)

    def comp_dyn(sl):
        @pl.when(sl == 0)
        def _():
            comp_s(0)
        @pl.when(sl == 1)
        def _():
            comp_s(1)

    def do_pass(p, _):
        lo = p * (NW * T_ROWS) + wid * T_ROWS
        for q in range(GB // 16 + 1):
            srl_v[pl.ds(q * 16, 16)] = jnp.zeros((16,), jnp.int32)

        def zacc(r, _):
            for j in range(XW // 16):
                acc_v[r, pl.ds(j * 16, 16)] = zeros16
            return 0
        lax.fori_loop(0, ACC_ROWS, zacc, 0)

        pltpu.async_copy(dst_hbm.at[pl.ds(0, SB)], dblk0_v, semd)
        pltpu.async_copy(src_hbm.at[pl.ds(0, SB)], sblk0_v, semd)

        def blk(b, st):
            rem, nf, pend = st
            def scan_into(dblk_v, sblk_v):
                pltpu.make_async_copy(dst_hbm.at[pl.ds(b * SB, SB)], dblk_v,
                                      semd).wait()
                pltpu.make_async_copy(src_hbm.at[pl.ds(b * SB, SB)], sblk_v,
                                      semd).wait()

                def scan(i, cnt):
                    ms, pks, ss, pcs = [], [], [], []
                    for u in range(4):
                        d16 = dblk_v[pl.ds((i * 4 + u) * 16, 16)]
                        s16 = sblk_v[pl.ds((i * 4 + u) * 16, 16)]
                        m = (d16 >= lo) & (d16 < lo + T_ROWS)
                        pk = (b * SB + (i * 4 + u) * 16 + iota) * 64 + (d16 - lo)
                        ms.append(m)
                        pks.append(pk)
                        ss.append(s16)
                        pcs.append(plsc.all_reduce_population_count(m))
                    off = cnt
                    for u in range(4):
                        plsc.store_compressed(pkl_v.at[pl.ds(off, 16)],
                                              pks[u], mask=ms[u])
                        plsc.store_compressed(srl_v.at[pl.ds(off, 16)],
                                              ss[u], mask=ms[u])
                        if u < 3:
                            off = off + pcs[u][0]
                    total = pcs[0] + pcs[1] + pcs[2] + pcs[3]
                    return cnt + total[0]
                return lax.fori_loop(0, SB // 64, scan, rem)

            @pl.when(b % 2 == 0)
            def _():
                @pl.when(b + 1 < NB)
                def _():
                    pltpu.async_copy(dst_hbm.at[pl.ds((b + 1) * SB, SB)],
                                     dblk1_v, semd)
                    pltpu.async_copy(src_hbm.at[pl.ds((b + 1) * SB, SB)],
                                     sblk1_v, semd)
            @pl.when(b % 2 == 1)
            def _():
                @pl.when(b + 1 < NB)
                def _():
                    pltpu.async_copy(dst_hbm.at[pl.ds((b + 1) * SB, SB)],
                                     dblk0_v, semd)
                    pltpu.async_copy(src_hbm.at[pl.ds((b + 1) * SB, SB)],
                                     sblk0_v, semd)
            avail0 = lax.cond(b % 2 == 0,
                              lambda: scan_into(dblk0_v, sblk0_v),
                              lambda: scan_into(dblk1_v, sblk1_v))

            nfull = lax.div(avail0, GB)

            def chunk(k, st2):
                nf2, pend2 = st2
                @pl.when(pend2 == 1)
                def _():
                    comp_dyn((nf2 + 1) % 2)
                fire_dyn(nf2 % 2, k * GB, GB)
                return (nf2 + 1, 1)
            nf, pend = lax.fori_loop(0, nfull, chunk, (nf, pend))

            rem2 = avail0 - nfull * GB
            tb = nfull * GB
            pq = pkl_v[pl.ds(tb, 16)]
            sq = srl_v[pl.ds(tb, 16)]
            pkl_v[pl.ds(0, 16)] = pq
            srl_v[pl.ds(0, 16)] = sq
            return (rem2, nf, pend)

        rem, nf, pend = lax.fori_loop(0, NB, blk, (0, 0, 0))

        @pl.when(pend == 1)
        def _():
            comp_dyn((nf + 1) % 2)

        @pl.when(rem > 0)
        def _():
            fire_dyn(nf % 2, 0, rem)
            comp_dyn(nf % 2)

        pltpu.sync_copy(acc_v.at[pl.ds(0, T_ROWS)],
                        num_hbm.at[pl.ds(lo, T_ROWS)])
        return 0

    lax.fori_loop(0, N_SLICES, do_pass, 0)


def _scb(xl, expv_flat, src, dst):
    mesh = plsc.VectorSubcoreMesh(core_axis_name="c", subcore_axis_name="s")
    f = functools.partial(
        pl.kernel,
        mesh=mesh,
        compiler_params=pltpu.CompilerParams(needs_layout_passes=False),
        out_type=jax.ShapeDtypeStruct((N_PAD, XW), jnp.float32),
        scratch_types=[
            pltpu.VMEM((SB,), jnp.int32),
            pltpu.VMEM((SB,), jnp.int32),
            pltpu.VMEM((SB,), jnp.int32),
            pltpu.VMEM((SB,), jnp.int32),
            pltpu.VMEM((SB + GB + 32,), jnp.int32),
            pltpu.VMEM((SB + GB + 32,), jnp.int32),
            pltpu.VMEM((GB + 16,), jnp.int32),
            pltpu.VMEM((GB + 16,), jnp.int32),
            pltpu.VMEM((GB,), jnp.int32),
            pltpu.VMEM((GB,), jnp.int32),
            pltpu.VMEM((4 * GB,), jnp.int32),
            pltpu.VMEM((4 * GB,), jnp.int32),
            pltpu.VMEM((4 * GB + 16,), jnp.float32),
            pltpu.VMEM((4 * GB + 16,), jnp.float32),
            pltpu.VMEM((GB, XW), jnp.float32),
            pltpu.VMEM((GB, XW), jnp.float32),
            pltpu.VMEM((ACC_ROWS, XW), jnp.float32),
            pltpu.SemaphoreType.DMA,
            pltpu.SemaphoreType.DMA,
            pltpu.SemaphoreType.DMA,
            pltpu.SemaphoreType.DMA,
            pltpu.SemaphoreType.DMA,
        ],
    )(_scb_body)
    return f(xl, expv_flat, src, dst)


def _tc3_body(num_ref, sl_ref, xl_ref, bg_ref, wf_ref, bf_ref, g_ref,
              b_ref, out_ref):
    cols = []
    for h in range(H):
        es = jnp.exp(jnp.minimum(sl_ref[:, h:h + 1], 60.0))
        znorm = num_ref[:, HF + 16 * h:HF + 16 * h + 1] + es + 1e-16
        seg = (num_ref[:, h * F_OUT:(h + 1) * F_OUT]
               + es * xl_ref[:, h * F_OUT:(h + 1) * F_OUT]) / znorm
        cols.append(seg)
    hg = jnp.concatenate(cols, axis=1) + bg_ref[...]
    hlin = jnp.dot(hg, wf_ref[...], preferred_element_type=jnp.float32) + bf_ref[...]
    mu = jnp.mean(hlin, axis=1, keepdims=True)
    var = jnp.mean((hlin - mu) ** 2, axis=1, keepdims=True)
    out_ref[...] = (hlin - mu) * lax.rsqrt(var + 1e-5) * g_ref[...] + b_ref[...]


def _tc3(num, sl, xl, bias_gat, W_fin, b_fin, ln_gamma, ln_beta):
    BN = 400
    return pl.pallas_call(
        _tc3_body,
        grid=(N // BN,),
        in_specs=[
            pl.BlockSpec((BN, XW), lambda i: (i, 0)),
            pl.BlockSpec((BN, 16), lambda i: (i, 0)),
            pl.BlockSpec((BN, XW), lambda i: (i, 0)),
            pl.BlockSpec((1, HF), lambda i: (0, 0)),
            pl.BlockSpec((HF, F_OUT), lambda i: (0, 0)),
            pl.BlockSpec((1, F_OUT), lambda i: (0, 0)),
            pl.BlockSpec((1, F_OUT), lambda i: (0, 0)),
            pl.BlockSpec((1, F_OUT), lambda i: (0, 0)),
        ],
        out_specs=pl.BlockSpec((BN, F_OUT), lambda i: (i, 0)),
        out_shape=jax.ShapeDtypeStruct((N, F_OUT), jnp.float32),
    )(num, sl, xl, bias_gat.reshape(1, HF), W_fin, b_fin.reshape(1, F_OUT),
      ln_gamma.reshape(1, F_OUT), ln_beta.reshape(1, F_OUT))


def kernel(X, E_idx, attr, W_l, b_l, W_r, b_r, att, bias_gat, W_fin, b_fin,
           ln_gamma, ln_beta):
    att_flat = att.reshape(HF)
    src = E_idx[0]
    dst = E_idx[1]
    xl, xr, sl = _tc1(X, W_l, b_l, W_r, b_r, att_flat)
    part = _sca(xl, xr, src, dst, att_flat)
    expv = _tc2(part)
    num = _scb(xl, expv.reshape(E * H), src, dst)
    h = _tc3(num[:N], sl, xl, bias_gat, W_fin, b_fin, ln_gamma, ln_beta)
    return (h, E_idx, attr)
